# Initial kernel scaffold; baseline (speedup 1.0000x reference)
import jax
import jax.numpy as jnp
from jax import lax
from jax.experimental import pallas as pl
from jax.experimental.pallas import tpu as pltpu
from jax.experimental.pallas import tpu_sc as plsc

N_NODES = 10000
N_EDGES = 320000
FDIM = 128
NC = 2
NS = 16
NW = NC * NS
CH = 128
CHUNKS = -(-N_EDGES // (NW * CH))
EW = CHUNKS * CH
E_PAD = EW * NW
ROWS_PER_TILE = N_NODES // NS
ROW_BLK = 125


def _ae_body(ea_ref, we0_ref, ae0_ref, we1_ref, ae1_ref, out_ref, csum_ref):
    ve0 = jnp.dot(we0_ref[...], ae0_ref[...], preferred_element_type=jnp.float32)
    ve1 = jnp.dot(we1_ref[...], ae1_ref[...], preferred_element_type=jnp.float32)
    ve = jnp.concatenate([ve0, ve1], axis=1)
    blk = jnp.dot(ea_ref[...], ve, preferred_element_type=jnp.float32)
    out_ref[...] = blk

    @pl.when(pl.program_id(0) == 0)
    def _():
        csum_ref[...] = jnp.zeros_like(csum_ref)

    csum_ref[...] += jnp.sum(blk, axis=0, keepdims=True)


def _run_ae(edge_attr, g0_We, g0_ae_col, g1_We, g1_ae_col):
    E = edge_attr.shape[0]
    BLK = 16000
    grid = (E // BLK,)
    return pl.pallas_call(
        _ae_body,
        grid=grid,
        in_specs=[
            pl.BlockSpec((BLK, edge_attr.shape[1]), lambda i: (i, 0)),
            pl.BlockSpec(g0_We.shape, lambda i: (0, 0)),
            pl.BlockSpec(g0_ae_col.shape, lambda i: (0, 0)),
            pl.BlockSpec(g1_We.shape, lambda i: (0, 0)),
            pl.BlockSpec(g1_ae_col.shape, lambda i: (0, 0)),
        ],
        out_specs=[
            pl.BlockSpec((BLK, 2), lambda i: (i, 0)),
            pl.BlockSpec((1, 2), lambda i: (0, 0)),
        ],
        out_shape=[
            jax.ShapeDtypeStruct((E, 2), jnp.float32),
            jax.ShapeDtypeStruct((1, 2), jnp.float32),
        ],
    )(edge_attr, g0_We, g0_ae_col, g1_We, g1_ae_col)


def _node_body(x_ref, w_ref, avt_ref, h_ref, sd_ref):
    h = jnp.dot(x_ref[...], w_ref[...], preferred_element_type=jnp.float32)
    h_ref[...] = h
    sd_ref[...] = jnp.dot(h, avt_ref[...], preferred_element_type=jnp.float32)


def _run_node(x, W, avT):
    BLK = 1000
    grid = (N_NODES // BLK,)
    return pl.pallas_call(
        _node_body,
        grid=grid,
        in_specs=[
            pl.BlockSpec((BLK, FDIM), lambda i: (i, 0)),
            pl.BlockSpec((FDIM, FDIM), lambda i: (0, 0)),
            pl.BlockSpec((FDIM, 2), lambda i: (0, 0)),
        ],
        out_specs=[
            pl.BlockSpec((BLK, FDIM), lambda i: (i, 0)),
            pl.BlockSpec((BLK, 2), lambda i: (i, 0)),
        ],
        out_shape=[
            jax.ShapeDtypeStruct((N_NODES, FDIM), jnp.float32),
            jax.ShapeDtypeStruct((N_NODES, 2), jnp.float32),
        ],
    )(x, W, avT)


def _mid_body(acc_ref, dpart_ref, sd_ref, csum_ref, h_ref, b_ref,
              w1_ref, av1t_ref, h1_ref, sd1_ref):
    cl = csum_ref[0, 0] * (1.0 / N_EDGES)
    s = sd_ref[:, 0:1]
    d = sd_ref[:, 1:2]
    al = s + d + cl
    al = jnp.where(al > 0, al, 0.2 * al)
    exl = jnp.exp(al)
    dsum = jnp.sum(dpart_ref[...], axis=0, keepdims=True)
    den = jnp.transpose(dsum) + exl
    rden = 1.0 / (den + 1e-16)
    h = h_ref[...]
    accs = acc_ref[0] + acc_ref[1]
    out = (accs + exl * h) * rden + b_ref[...]
    x1 = jnp.maximum(out, 0.0)
    h1 = jnp.dot(x1, w1_ref[...], preferred_element_type=jnp.float32)
    h1_ref[...] = h1
    sd1_ref[...] = jnp.dot(h1, av1t_ref[...], preferred_element_type=jnp.float32)


def _run_mid(acc, dpart, sd, csum, h, b_row, W1, av1T):
    BLK = 1000
    grid = (N_NODES // BLK,)
    return pl.pallas_call(
        _mid_body,
        grid=grid,
        in_specs=[
            pl.BlockSpec((2, BLK, FDIM), lambda i: (0, i, 0)),
            pl.BlockSpec((NW, BLK), lambda i: (0, i)),
            pl.BlockSpec((BLK, 2), lambda i: (i, 0)),
            pl.BlockSpec((1, 2), lambda i: (0, 0)),
            pl.BlockSpec((BLK, FDIM), lambda i: (i, 0)),
            pl.BlockSpec((1, FDIM), lambda i: (0, 0)),
            pl.BlockSpec((FDIM, FDIM), lambda i: (0, 0)),
            pl.BlockSpec((FDIM, 2), lambda i: (0, 0)),
        ],
        out_specs=[
            pl.BlockSpec((BLK, FDIM), lambda i: (i, 0)),
            pl.BlockSpec((BLK, 2), lambda i: (i, 0)),
        ],
        out_shape=[
            jax.ShapeDtypeStruct((N_NODES, FDIM), jnp.float32),
            jax.ShapeDtypeStruct((N_NODES, 2), jnp.float32),
        ],
    )(acc, dpart, sd, csum, h, b_row, W1, av1T)


def _head_body(acc_ref, dpart_ref, sd_ref, csum_ref, h_ref, b_ref, batch_ref,
               lin0w_ref, lin0b_ref, h0w_ref, h0b_ref, h1w_ref, h1b_ref,
               out_ref):
    cl = csum_ref[0, 1] * (1.0 / N_EDGES)
    s = sd_ref[:, 0:1]
    d = sd_ref[:, 1:2]
    al = s + d + cl
    al = jnp.where(al > 0, al, 0.2 * al)
    exl = jnp.exp(al)
    dsum = jnp.sum(dpart_ref[...], axis=0, keepdims=True)
    den = jnp.transpose(dsum) + exl
    rden = 1.0 / (den + 1e-16)
    h = h_ref[...]
    hf = (acc_ref[0] + acc_ref[1] + exl * h) * rden + b_ref[...]

    batch = batch_ref[...]
    gids = lax.broadcasted_iota(jnp.int32, (1, 16), 1)
    counts = jnp.sum((batch < gids).astype(jnp.float32), axis=0,
                     keepdims=True)
    node_iota = lax.broadcasted_iota(jnp.float32, (16, N_NODES), 1)
    onehot = (node_iota == jnp.transpose(counts)).astype(jnp.float32)
    z = jnp.dot(onehot, hf, preferred_element_type=jnp.float32)

    z = jnp.maximum(jnp.dot(z, lin0w_ref[...],
                            preferred_element_type=jnp.float32) + lin0b_ref[...], 0.0)
    z = jnp.maximum(jnp.dot(z, h0w_ref[...],
                            preferred_element_type=jnp.float32) + h0b_ref[...], 0.0)
    out_ref[...] = jnp.dot(z, h1w_ref[...],
                           preferred_element_type=jnp.float32) + h1b_ref[...]


def _run_head(acc, dpart, sd, csum, h, b_row, batch_col,
              lin0_W, lin0_b, h0_W, h0_b, h1_W, h1_b):
    return pl.pallas_call(
        _head_body,
        out_shape=jax.ShapeDtypeStruct((16, 16), jnp.float32),
    )(acc, dpart, sd, csum, h, b_row, batch_col,
      lin0_W, lin0_b.reshape(1, -1), h0_W, h0_b.reshape(1, -1),
      h1_W, h1_b.reshape(1, -1))


def _sc_edge_body(h_hbm, s_hbm, d_hbm, ae_hbm, src_hbm, dst_hbm,
                  acc_hbm, dpart_hbm,
                  s_v, d_v, den_v, src_v, dst_v, ae_v, ex_v, rows_v,
                  acc_sh, sem):
    cid = lax.axis_index("c")
    sid = lax.axis_index("s")
    wid = cid * NS + sid

    pltpu.sync_copy(s_hbm, s_v)
    pltpu.sync_copy(d_hbm, d_v)

    zero16 = jnp.zeros((16,), jnp.float32)

    def _zero_den(i, carry):
        den_v[pl.ds(i * 16, 16)] = zero16
        return carry
    lax.fori_loop(0, N_NODES // 16, _zero_den, 0)

    def _zero_rows(i, carry):
        for k in range(FDIM // 16):
            rows_v[i, pl.ds(k * 16, 16)] = zero16
        return carry
    lax.fori_loop(0, CH, _zero_rows, 0)

    for j in range(ROWS_PER_TILE // ROW_BLK):
        pltpu.sync_copy(rows_v.at[pl.ds(0, ROW_BLK)],
                        acc_sh.at[pl.ds(sid * ROWS_PER_TILE + j * ROW_BLK, ROW_BLK)])
    plsc.subcore_barrier()

    def _chunk(it, carry):
        base = wid * EW + it * CH
        pltpu.sync_copy(src_hbm.at[pl.ds(base, CH)], src_v)
        pltpu.sync_copy(dst_hbm.at[pl.ds(base, CH)], dst_v)
        pltpu.sync_copy(ae_hbm.at[pl.ds(base, CH)], ae_v)
        pltpu.async_copy(h_hbm.at[src_v], rows_v, sem).wait()

        for g in range(CH // 16):
            sl = pl.ds(g * 16, 16)
            sv = src_v[sl]
            dv = dst_v[sl]
            a = (plsc.load_gather(s_v, [sv]) + plsc.load_gather(d_v, [dv])
                 + ae_v[sl])
            a = jnp.where(a > 0, a, 0.2 * a)
            ex = jnp.exp(a)
            plsc.addupdate_scatter(den_v, [dv], ex)
            ex_v[sl] = ex

        def _scale(e, carry2):
            ce = jnp.broadcast_to(ex_v[e], (16,))
            for k in range(FDIM // 16):
                ksl = pl.ds(k * 16, 16)
                rows_v[e, ksl] = rows_v[e, ksl] * ce
            return carry2
        lax.fori_loop(0, CH, _scale, 0)

        pltpu.sync_copy(rows_v, acc_sh.at[dst_v], add=True)
        return carry

    lax.fori_loop(0, CHUNKS, _chunk, 0)

    pltpu.sync_copy(den_v, dpart_hbm.at[wid])
    plsc.subcore_barrier()
    pltpu.sync_copy(acc_sh.at[pl.ds(sid * ROWS_PER_TILE, ROWS_PER_TILE)],
                    acc_hbm.at[cid, pl.ds(sid * ROWS_PER_TILE, ROWS_PER_TILE)])


def _run_sc_edge(h, s, d, ae, src, dst):
    mesh = plsc.VectorSubcoreMesh(core_axis_name="c", subcore_axis_name="s")
    fn = pl.kernel(
        _sc_edge_body,
        out_type=[
            jax.ShapeDtypeStruct((NC, N_NODES, FDIM), jnp.float32),
            jax.ShapeDtypeStruct((NW, N_NODES), jnp.float32),
        ],
        mesh=mesh,
        scratch_types=[
            pltpu.VMEM((N_NODES,), jnp.float32),
            pltpu.VMEM((N_NODES,), jnp.float32),
            pltpu.VMEM((N_NODES,), jnp.float32),
            pltpu.VMEM((CH,), jnp.int32),
            pltpu.VMEM((CH,), jnp.int32),
            pltpu.VMEM((CH,), jnp.float32),
            pltpu.VMEM((CH,), jnp.float32),
            pltpu.VMEM((CH, FDIM), jnp.float32),
            pltpu.VMEM_SHARED((N_NODES, FDIM), jnp.float32),
            pltpu.SemaphoreType.DMA,
        ],
    )
    return fn(h, s, d, ae, src, dst)


def kernel(x, edge_index, batch, edge_attr,
           g0_W, g0_asrc, g0_adst, g0_We, g0_ae, g0_b,
           g1_W, g1_asrc, g1_adst, g1_We, g1_ae, g1_b,
           lin0_W, lin0_b, h0_W, h0_b, h1_W, h1_b):
    N = x.shape[0]
    E = edge_index.shape[1]

    av0T = jnp.stack([g0_asrc[0], g0_adst[0]], axis=1)
    av1T = jnp.stack([g1_asrc[0], g1_adst[0]], axis=1)

    AE, csum = _run_ae(edge_attr, g0_We, g0_ae.reshape(-1, 1),
                       g1_We, g1_ae.reshape(-1, 1))

    pad = E_PAD - E
    pad_idx = (jnp.arange(pad, dtype=jnp.int32) % N)
    src_p = jnp.concatenate([edge_index[0], pad_idx])
    dst_p = jnp.concatenate([edge_index[1], pad_idx])
    neg = jnp.full((pad,), -1e30, jnp.float32)
    ae0_p = jnp.concatenate([AE[:, 0], neg])
    ae1_p = jnp.concatenate([AE[:, 1], neg])

    h0, sd0 = _run_node(x, g0_W, av0T)
    acc0, dpart0 = _run_sc_edge(h0, sd0[:, 0], sd0[:, 1], ae0_p, src_p, dst_p)
    h1, sd1 = _run_mid(acc0, dpart0, sd0, csum, h0, g0_b.reshape(1, -1),
                       g1_W, av1T)
    acc1, dpart1 = _run_sc_edge(h1, sd1[:, 0], sd1[:, 1], ae1_p, src_p, dst_p)
    out = _run_head(acc1, dpart1, sd1, csum, h1, g1_b.reshape(1, -1),
                    batch.reshape(-1, 1),
                    lin0_W, lin0_b, h0_W, h0_b, h1_W, h1_b)
    return out


# speedup vs baseline: 17.5310x; 17.5310x over previous
import jax
import jax.numpy as jnp
from jax import lax
from jax.experimental import pallas as pl
from jax.experimental.pallas import tpu as pltpu
from jax.experimental.pallas import tpu_sc as plsc

N_NODES = 10000
NP = 10240
N_EDGES = 320000
FDIM = 128
NC = 2
NS = 16
NW = NC * NS
CH = 128
CHUNKS = -(-N_EDGES // (NW * CH))
EW = CHUNKS * CH
E_PAD = EW * NW
ROWS_PER_TILE = NP // NS
ROW_BLK = 128
NBLK = 2048


def _ae_body(ea_ref, we0_ref, ae0_ref, we1_ref, ae1_ref, out_ref, csum_ref):
    ve0 = jnp.dot(we0_ref[...], ae0_ref[...], preferred_element_type=jnp.float32)
    ve1 = jnp.dot(we1_ref[...], ae1_ref[...], preferred_element_type=jnp.float32)
    ve = jnp.concatenate([ve0, ve1], axis=1)
    blk = jnp.dot(ea_ref[...], ve, preferred_element_type=jnp.float32)
    out_ref[...] = blk

    @pl.when(pl.program_id(0) == 0)
    def _():
        csum_ref[...] = jnp.zeros_like(csum_ref)

    csum_ref[...] += jnp.sum(blk, axis=0, keepdims=True)


def _run_ae(edge_attr, g0_We, g0_ae_col, g1_We, g1_ae_col):
    E = edge_attr.shape[0]
    BLK = 16000
    grid = (E // BLK,)
    return pl.pallas_call(
        _ae_body,
        grid=grid,
        in_specs=[
            pl.BlockSpec((BLK, edge_attr.shape[1]), lambda i: (i, 0)),
            pl.BlockSpec(g0_We.shape, lambda i: (0, 0)),
            pl.BlockSpec(g0_ae_col.shape, lambda i: (0, 0)),
            pl.BlockSpec(g1_We.shape, lambda i: (0, 0)),
            pl.BlockSpec(g1_ae_col.shape, lambda i: (0, 0)),
        ],
        out_specs=[
            pl.BlockSpec((BLK, 2), lambda i: (i, 0)),
            pl.BlockSpec((1, 2), lambda i: (0, 0)),
        ],
        out_shape=[
            jax.ShapeDtypeStruct((E, 2), jnp.float32),
            jax.ShapeDtypeStruct((1, 2), jnp.float32),
        ],
    )(edge_attr, g0_We, g0_ae_col, g1_We, g1_ae_col)


def _node_body(x_ref, w_ref, avt_ref, h_ref, sd_ref):
    h = jnp.dot(x_ref[...], w_ref[...], preferred_element_type=jnp.float32)
    h_ref[...] = h
    sd_ref[...] = jnp.dot(h, avt_ref[...], preferred_element_type=jnp.float32)


def _run_node(x, W, avT):
    BLK = NBLK
    grid = (NP // BLK,)
    return pl.pallas_call(
        _node_body,
        grid=grid,
        in_specs=[
            pl.BlockSpec((BLK, FDIM), lambda i: (i, 0)),
            pl.BlockSpec((FDIM, FDIM), lambda i: (0, 0)),
            pl.BlockSpec((FDIM, 2), lambda i: (0, 0)),
        ],
        out_specs=[
            pl.BlockSpec((BLK, FDIM), lambda i: (i, 0)),
            pl.BlockSpec((BLK, 2), lambda i: (i, 0)),
        ],
        out_shape=[
            jax.ShapeDtypeStruct((NP, FDIM), jnp.float32),
            jax.ShapeDtypeStruct((NP, 2), jnp.float32),
        ],
    )(x, W, avT)


def _mid_body(acc_ref, dpart_ref, sd_ref, csum_ref, h_ref, b_ref,
              w1_ref, av1t_ref, h1_ref, sd1_ref):
    cl = csum_ref[0, 0] * (1.0 / N_EDGES)
    s = sd_ref[:, 0:1]
    d = sd_ref[:, 1:2]
    al = s + d + cl
    al = jnp.where(al > 0, al, 0.2 * al)
    exl = jnp.exp(al)
    ones = jnp.ones((NW, 1), jnp.float32)
    dsum = lax.dot_general(dpart_ref[...], ones, (((0,), (0,)), ((), ())),
                           preferred_element_type=jnp.float32)
    den = dsum + exl
    rden = 1.0 / (den + 1e-16)
    h = h_ref[...]
    accs = acc_ref[0] + acc_ref[1]
    out = (accs + exl * h) * rden + b_ref[...]
    x1 = jnp.maximum(out, 0.0)
    h1 = jnp.dot(x1, w1_ref[...], preferred_element_type=jnp.float32)
    h1_ref[...] = h1
    sd1_ref[...] = jnp.dot(h1, av1t_ref[...], preferred_element_type=jnp.float32)


def _run_mid(acc, dpart, sd, csum, h, b_row, W1, av1T):
    BLK = NBLK
    grid = (NP // BLK,)
    return pl.pallas_call(
        _mid_body,
        grid=grid,
        in_specs=[
            pl.BlockSpec((2, BLK, FDIM), lambda i: (0, i, 0)),
            pl.BlockSpec((NW, BLK), lambda i: (0, i)),
            pl.BlockSpec((BLK, 2), lambda i: (i, 0)),
            pl.BlockSpec((1, 2), lambda i: (0, 0)),
            pl.BlockSpec((BLK, FDIM), lambda i: (i, 0)),
            pl.BlockSpec((1, FDIM), lambda i: (0, 0)),
            pl.BlockSpec((FDIM, FDIM), lambda i: (0, 0)),
            pl.BlockSpec((FDIM, 2), lambda i: (0, 0)),
        ],
        out_specs=[
            pl.BlockSpec((BLK, FDIM), lambda i: (i, 0)),
            pl.BlockSpec((BLK, 2), lambda i: (i, 0)),
        ],
        out_shape=[
            jax.ShapeDtypeStruct((NP, FDIM), jnp.float32),
            jax.ShapeDtypeStruct((NP, 2), jnp.float32),
        ],
    )(acc, dpart, sd, csum, h, b_row, W1, av1T)


def _head_body(acc_ref, dpart_ref, sd_ref, csum_ref, h_ref, b_ref, batch_ref,
               lin0w_ref, lin0b_ref, h0w_ref, h0b_ref, h1w_ref, h1b_ref,
               out_ref):
    cl = csum_ref[0, 1] * (1.0 / N_EDGES)
    s = sd_ref[:, 0:1]
    d = sd_ref[:, 1:2]
    al = s + d + cl
    al = jnp.where(al > 0, al, 0.2 * al)
    exl = jnp.exp(al)
    ones = jnp.ones((NW, 1), jnp.float32)
    dsum = lax.dot_general(dpart_ref[...], ones, (((0,), (0,)), ((), ())),
                           preferred_element_type=jnp.float32)
    den = dsum + exl
    rden = 1.0 / (den + 1e-16)
    h = h_ref[...]
    hf = (acc_ref[0] + acc_ref[1] + exl * h) * rden + b_ref[...]

    batch = batch_ref[...]
    gids = lax.broadcasted_iota(jnp.int32, (1, 16), 1)
    lt = (batch < gids).astype(jnp.float32)
    onesn = jnp.ones((NP, 1), jnp.float32)
    counts_f = lax.dot_general(lt, onesn, (((0,), (0,)), ((), ())),
                               preferred_element_type=jnp.float32)
    counts = counts_f.astype(jnp.int32)
    node_iota = lax.broadcasted_iota(jnp.int32, (16, NP), 1)
    onehot = (node_iota == counts).astype(jnp.float32)
    z = jnp.dot(onehot, hf, preferred_element_type=jnp.float32)

    z = jnp.maximum(jnp.dot(z, lin0w_ref[...],
                            preferred_element_type=jnp.float32) + lin0b_ref[...], 0.0)
    z = jnp.maximum(jnp.dot(z, h0w_ref[...],
                            preferred_element_type=jnp.float32) + h0b_ref[...], 0.0)
    out_ref[...] = jnp.dot(z, h1w_ref[...],
                           preferred_element_type=jnp.float32) + h1b_ref[...]


def _run_head(acc, dpart, sd, csum, h, b_row, batch_col,
              lin0_W, lin0_b, h0_W, h0_b, h1_W, h1_b):
    return pl.pallas_call(
        _head_body,
        out_shape=jax.ShapeDtypeStruct((16, 16), jnp.float32),
    )(acc, dpart, sd, csum, h, b_row, batch_col,
      lin0_W, lin0_b.reshape(1, -1), h0_W, h0_b.reshape(1, -1),
      h1_W, h1_b.reshape(1, -1))


def _sc_edge_body(h_hbm, s_hbm, d_hbm, ae_hbm, src_hbm, dst_hbm,
                  acc_hbm, dpart_hbm,
                  s_v, d_v, den_v, src_v, dst_v, ae_v, ex_v, rows_v,
                  acc_sh, sem):
    cid = lax.axis_index("c")
    sid = lax.axis_index("s")
    wid = cid * NS + sid

    pltpu.sync_copy(s_hbm, s_v)
    pltpu.sync_copy(d_hbm, d_v)

    zero16 = jnp.zeros((16,), jnp.float32)

    def _zero_den(i, carry):
        den_v[pl.ds(i * 16, 16)] = zero16
        return carry
    lax.fori_loop(0, NP // 16, _zero_den, 0)

    def _zero_rows(i, carry):
        for k in range(FDIM // 16):
            rows_v[i, pl.ds(k * 16, 16)] = zero16
        return carry
    lax.fori_loop(0, CH, _zero_rows, 0)

    for j in range(ROWS_PER_TILE // ROW_BLK):
        pltpu.sync_copy(rows_v.at[pl.ds(0, ROW_BLK)],
                        acc_sh.at[pl.ds(sid * ROWS_PER_TILE + j * ROW_BLK, ROW_BLK)])
    plsc.subcore_barrier()

    def _chunk(it, carry):
        base = wid * EW + it * CH
        pltpu.sync_copy(src_hbm.at[pl.ds(base, CH)], src_v)
        pltpu.sync_copy(dst_hbm.at[pl.ds(base, CH)], dst_v)
        pltpu.sync_copy(ae_hbm.at[pl.ds(base, CH)], ae_v)
        pltpu.async_copy(h_hbm.at[src_v], rows_v, sem).wait()

        for g in range(CH // 16):
            sl = pl.ds(g * 16, 16)
            sv = src_v[sl]
            dv = dst_v[sl]
            a = (plsc.load_gather(s_v, [sv]) + plsc.load_gather(d_v, [dv])
                 + ae_v[sl])
            a = jnp.where(a > 0, a, 0.2 * a)
            ex = jnp.exp(a)
            plsc.addupdate_scatter(den_v, [dv], ex)
            ex_v[sl] = ex

        def _scale(e, carry2):
            eidx = jnp.broadcast_to(e, (16,)).astype(jnp.int32)
            ce = plsc.load_gather(ex_v, [eidx])
            for k in range(FDIM // 16):
                ksl = pl.ds(k * 16, 16)
                rows_v[e, ksl] = rows_v[e, ksl] * ce
            return carry2
        lax.fori_loop(0, CH, _scale, 0)

        pltpu.sync_copy(rows_v, acc_sh.at[dst_v], add=True)
        return carry

    lax.fori_loop(0, CHUNKS, _chunk, 0)

    pltpu.sync_copy(den_v, dpart_hbm.at[wid])
    plsc.subcore_barrier()
    pltpu.sync_copy(acc_sh.at[pl.ds(sid * ROWS_PER_TILE, ROWS_PER_TILE)],
                    acc_hbm.at[cid, pl.ds(sid * ROWS_PER_TILE, ROWS_PER_TILE)])


def _run_sc_edge(h, s, d, ae, src, dst):
    mesh = plsc.VectorSubcoreMesh(core_axis_name="c", subcore_axis_name="s")
    fn = pl.kernel(
        _sc_edge_body,
        out_type=[
            jax.ShapeDtypeStruct((NC, NP, FDIM), jnp.float32),
            jax.ShapeDtypeStruct((NW, NP), jnp.float32),
        ],
        mesh=mesh,
        compiler_params=pltpu.CompilerParams(needs_layout_passes=False),
        scratch_types=[
            pltpu.VMEM((NP,), jnp.float32),
            pltpu.VMEM((NP,), jnp.float32),
            pltpu.VMEM((NP,), jnp.float32),
            pltpu.VMEM((CH,), jnp.int32),
            pltpu.VMEM((CH,), jnp.int32),
            pltpu.VMEM((CH,), jnp.float32),
            pltpu.VMEM((CH,), jnp.float32),
            pltpu.VMEM((CH, FDIM), jnp.float32),
            pltpu.VMEM_SHARED((NP, FDIM), jnp.float32),
            pltpu.SemaphoreType.DMA,
        ],
    )
    return fn(h, s, d, ae, src, dst)


def kernel(x, edge_index, batch, edge_attr,
           g0_W, g0_asrc, g0_adst, g0_We, g0_ae, g0_b,
           g1_W, g1_asrc, g1_adst, g1_We, g1_ae, g1_b,
           lin0_W, lin0_b, h0_W, h0_b, h1_W, h1_b):
    N = x.shape[0]
    E = edge_index.shape[1]

    av0T = jnp.stack([g0_asrc[0], g0_adst[0]], axis=1)
    av1T = jnp.stack([g1_asrc[0], g1_adst[0]], axis=1)

    AE, csum = _run_ae(edge_attr, g0_We, g0_ae.reshape(-1, 1),
                       g1_We, g1_ae.reshape(-1, 1))

    pad = E_PAD - E
    pad_idx = (jnp.arange(pad, dtype=jnp.int32) % N)
    src_p = jnp.concatenate([edge_index[0], pad_idx])
    dst_p = jnp.concatenate([edge_index[1], pad_idx])
    neg = jnp.full((pad,), -1e30, jnp.float32)
    ae0_p = jnp.concatenate([AE[:, 0], neg])
    ae1_p = jnp.concatenate([AE[:, 1], neg])

    xp = jnp.concatenate([x, jnp.zeros((NP - N, FDIM), jnp.float32)], axis=0)
    batch_p = jnp.concatenate(
        [batch, jnp.full((NP - N,), 16, batch.dtype)]).reshape(-1, 1)

    h0, sd0 = _run_node(xp, g0_W, av0T)
    acc0, dpart0 = _run_sc_edge(h0, sd0[:, 0], sd0[:, 1], ae0_p, src_p, dst_p)
    h1, sd1 = _run_mid(acc0, dpart0, sd0, csum, h0, g0_b.reshape(1, -1),
                       g1_W, av1T)
    acc1, dpart1 = _run_sc_edge(h1, sd1[:, 0], sd1[:, 1], ae1_p, src_p, dst_p)
    out = _run_head(acc1, dpart1, sd1, csum, h1, g1_b.reshape(1, -1),
                    batch_p,
                    lin0_W, lin0_b, h0_W, h0_b, h1_W, h1_b)
    return out


# speedup vs baseline: 19.4196x; 1.1077x over previous
import jax
import jax.numpy as jnp
from jax import lax
from jax.experimental import pallas as pl
from jax.experimental.pallas import tpu as pltpu
from jax.experimental.pallas import tpu_sc as plsc

N_NODES = 10000
NP = 10240
N_EDGES = 320000
FDIM = 128
NC = 2
NS = 16
NW = NC * NS
CH = 64
CHUNKS = 158
EW = CHUNKS * CH
E_PAD = EW * NW
ROWS_PER_TILE = NP // NS
ROW_BLK = 64
NBLK = 2048


def _ae_body(ea_ref, we0_ref, ae0_ref, we1_ref, ae1_ref, out_ref, csum_ref):
    ve0 = jnp.dot(we0_ref[...], ae0_ref[...], preferred_element_type=jnp.float32)
    ve1 = jnp.dot(we1_ref[...], ae1_ref[...], preferred_element_type=jnp.float32)
    ve = jnp.concatenate([ve0, ve1], axis=1)
    blk = jnp.dot(ea_ref[...], ve, preferred_element_type=jnp.float32)
    out_ref[...] = blk

    @pl.when(pl.program_id(0) == 0)
    def _():
        csum_ref[...] = jnp.zeros_like(csum_ref)

    csum_ref[...] += jnp.sum(blk, axis=0, keepdims=True)


def _run_ae(edge_attr, g0_We, g0_ae_col, g1_We, g1_ae_col):
    E = edge_attr.shape[0]
    BLK = 16000
    grid = (E // BLK,)
    return pl.pallas_call(
        _ae_body,
        grid=grid,
        in_specs=[
            pl.BlockSpec((BLK, edge_attr.shape[1]), lambda i: (i, 0)),
            pl.BlockSpec(g0_We.shape, lambda i: (0, 0)),
            pl.BlockSpec(g0_ae_col.shape, lambda i: (0, 0)),
            pl.BlockSpec(g1_We.shape, lambda i: (0, 0)),
            pl.BlockSpec(g1_ae_col.shape, lambda i: (0, 0)),
        ],
        out_specs=[
            pl.BlockSpec((BLK, 2), lambda i: (i, 0)),
            pl.BlockSpec((1, 2), lambda i: (0, 0)),
        ],
        out_shape=[
            jax.ShapeDtypeStruct((E, 2), jnp.float32),
            jax.ShapeDtypeStruct((1, 2), jnp.float32),
        ],
    )(edge_attr, g0_We, g0_ae_col, g1_We, g1_ae_col)


def _node_body(x_ref, w_ref, avt_ref, h_ref, sd_ref):
    h = jnp.dot(x_ref[...], w_ref[...], preferred_element_type=jnp.float32)
    h_ref[...] = h
    sd_ref[...] = jnp.dot(h, avt_ref[...], preferred_element_type=jnp.float32)


def _run_node(x, W, avT):
    BLK = NBLK
    grid = (NP // BLK,)
    return pl.pallas_call(
        _node_body,
        grid=grid,
        in_specs=[
            pl.BlockSpec((BLK, FDIM), lambda i: (i, 0)),
            pl.BlockSpec((FDIM, FDIM), lambda i: (0, 0)),
            pl.BlockSpec((FDIM, 2), lambda i: (0, 0)),
        ],
        out_specs=[
            pl.BlockSpec((BLK, FDIM), lambda i: (i, 0)),
            pl.BlockSpec((BLK, 2), lambda i: (i, 0)),
        ],
        out_shape=[
            jax.ShapeDtypeStruct((NP, FDIM), jnp.float32),
            jax.ShapeDtypeStruct((NP, 2), jnp.float32),
        ],
    )(x, W, avT)


def _mid_body(acc_ref, dpart_ref, sd_ref, csum_ref, h_ref, b_ref,
              w1_ref, av1t_ref, h1_ref, sd1_ref):
    cl = csum_ref[0, 0] * (1.0 / N_EDGES)
    s = sd_ref[:, 0:1]
    d = sd_ref[:, 1:2]
    al = s + d + cl
    al = jnp.where(al > 0, al, 0.2 * al)
    exl = jnp.exp(al)
    ones = jnp.ones((NW, 1), jnp.float32)
    dsum = lax.dot_general(dpart_ref[...], ones, (((0,), (0,)), ((), ())),
                           preferred_element_type=jnp.float32)
    den = dsum + exl
    rden = 1.0 / (den + 1e-16)
    h = h_ref[...]
    accs = acc_ref[0] + acc_ref[1]
    out = (accs + exl * h) * rden + b_ref[...]
    x1 = jnp.maximum(out, 0.0)
    h1 = jnp.dot(x1, w1_ref[...], preferred_element_type=jnp.float32)
    h1_ref[...] = h1
    sd1_ref[...] = jnp.dot(h1, av1t_ref[...], preferred_element_type=jnp.float32)


def _run_mid(acc, dpart, sd, csum, h, b_row, W1, av1T):
    BLK = NBLK
    grid = (NP // BLK,)
    return pl.pallas_call(
        _mid_body,
        grid=grid,
        in_specs=[
            pl.BlockSpec((2, BLK, FDIM), lambda i: (0, i, 0)),
            pl.BlockSpec((NW, BLK), lambda i: (0, i)),
            pl.BlockSpec((BLK, 2), lambda i: (i, 0)),
            pl.BlockSpec((1, 2), lambda i: (0, 0)),
            pl.BlockSpec((BLK, FDIM), lambda i: (i, 0)),
            pl.BlockSpec((1, FDIM), lambda i: (0, 0)),
            pl.BlockSpec((FDIM, FDIM), lambda i: (0, 0)),
            pl.BlockSpec((FDIM, 2), lambda i: (0, 0)),
        ],
        out_specs=[
            pl.BlockSpec((BLK, FDIM), lambda i: (i, 0)),
            pl.BlockSpec((BLK, 2), lambda i: (i, 0)),
        ],
        out_shape=[
            jax.ShapeDtypeStruct((NP, FDIM), jnp.float32),
            jax.ShapeDtypeStruct((NP, 2), jnp.float32),
        ],
    )(acc, dpart, sd, csum, h, b_row, W1, av1T)


def _head_body(acc_ref, dpart_ref, sd_ref, csum_ref, h_ref, b_ref, batch_ref,
               lin0w_ref, lin0b_ref, h0w_ref, h0b_ref, h1w_ref, h1b_ref,
               out_ref):
    cl = csum_ref[0, 1] * (1.0 / N_EDGES)
    s = sd_ref[:, 0:1]
    d = sd_ref[:, 1:2]
    al = s + d + cl
    al = jnp.where(al > 0, al, 0.2 * al)
    exl = jnp.exp(al)
    ones = jnp.ones((NW, 1), jnp.float32)
    dsum = lax.dot_general(dpart_ref[...], ones, (((0,), (0,)), ((), ())),
                           preferred_element_type=jnp.float32)
    den = dsum + exl
    rden = 1.0 / (den + 1e-16)
    h = h_ref[...]
    hf = (acc_ref[0] + acc_ref[1] + exl * h) * rden + b_ref[...]

    batch = batch_ref[...]
    gids = lax.broadcasted_iota(jnp.int32, (1, 16), 1)
    lt = (batch < gids).astype(jnp.float32)
    onesn = jnp.ones((NP, 1), jnp.float32)
    counts_f = lax.dot_general(lt, onesn, (((0,), (0,)), ((), ())),
                               preferred_element_type=jnp.float32)
    counts = counts_f.astype(jnp.int32)
    node_iota = lax.broadcasted_iota(jnp.int32, (16, NP), 1)
    onehot = (node_iota == counts).astype(jnp.float32)
    z = jnp.dot(onehot, hf, preferred_element_type=jnp.float32)

    z = jnp.maximum(jnp.dot(z, lin0w_ref[...],
                            preferred_element_type=jnp.float32) + lin0b_ref[...], 0.0)
    z = jnp.maximum(jnp.dot(z, h0w_ref[...],
                            preferred_element_type=jnp.float32) + h0b_ref[...], 0.0)
    out_ref[...] = jnp.dot(z, h1w_ref[...],
                           preferred_element_type=jnp.float32) + h1b_ref[...]


def _run_head(acc, dpart, sd, csum, h, b_row, batch_col,
              lin0_W, lin0_b, h0_W, h0_b, h1_W, h1_b):
    return pl.pallas_call(
        _head_body,
        out_shape=jax.ShapeDtypeStruct((16, 16), jnp.float32),
    )(acc, dpart, sd, csum, h, b_row, batch_col,
      lin0_W, lin0_b.reshape(1, -1), h0_W, h0_b.reshape(1, -1),
      h1_W, h1_b.reshape(1, -1))


def _sc_edge_body(h_hbm, s_hbm, d_hbm, ae_hbm, src_hbm, dst_hbm,
                  acc_hbm, dpart_hbm,
                  s_v, d_v, den_v,
                  src_v0, dst_v0, ae_v0, rows_v0,
                  src_v1, dst_v1, ae_v1, rows_v1,
                  ex_v, acc_sh, sem0, sem1):
    cid = lax.axis_index("c")
    sid = lax.axis_index("s")
    wid = cid * NS + sid
    wbase = wid * EW

    src_b = (src_v0, src_v1)
    dst_b = (dst_v0, dst_v1)
    ae_b = (ae_v0, ae_v1)
    rows_b = (rows_v0, rows_v1)
    sems = (sem0, sem1)

    pltpu.sync_copy(s_hbm, s_v)
    pltpu.sync_copy(d_hbm, d_v)

    zero16 = jnp.zeros((16,), jnp.float32)

    @plsc.parallel_loop(0, NP // 16, unroll=4)
    def _(i):
        den_v[pl.ds(i * 16, 16)] = zero16

    @plsc.parallel_loop(0, CH, unroll=4)
    def _(i):
        for k in range(FDIM // 16):
            rows_v0[i, pl.ds(k * 16, 16)] = zero16

    for j in range(ROWS_PER_TILE // ROW_BLK):
        pltpu.sync_copy(rows_v0.at[pl.ds(0, ROW_BLK)],
                        acc_sh.at[pl.ds(sid * ROWS_PER_TILE + j * ROW_BLK, ROW_BLK)])
    plsc.subcore_barrier()

    def _stage(i, b):
        base = wbase + i * CH
        pltpu.sync_copy(src_hbm.at[pl.ds(base, CH)], src_b[b])
        pltpu.sync_copy(dst_hbm.at[pl.ds(base, CH)], dst_b[b])
        pltpu.sync_copy(ae_hbm.at[pl.ds(base, CH)], ae_b[b])
        pltpu.async_copy(h_hbm.at[src_b[b]], rows_b[b], sems[b])

    _stage(0, 0)

    def _pair(it, carry):
        for b in range(2):
            i = it * 2 + b
            nxt = i + 1

            @pl.when(nxt < CHUNKS)
            def _():
                _stage(nxt, 1 - b)

            pltpu.make_async_copy(h_hbm.at[src_b[b]], rows_b[b], sems[b]).wait()

            rows_v = rows_b[b]
            for g in range(CH // 16):
                sl = pl.ds(g * 16, 16)
                sv = src_b[b][sl]
                dv = dst_b[b][sl]
                a = (plsc.load_gather(s_v, [sv]) + plsc.load_gather(d_v, [dv])
                     + ae_b[b][sl])
                a = jnp.where(a > 0, a, 0.2 * a)
                ex = jnp.exp(a)
                plsc.addupdate_scatter(den_v, [dv], ex)
                ex_v[sl] = ex

            @plsc.parallel_loop(0, CH, unroll=4)
            def _(e):
                eidx = jnp.broadcast_to(e, (16,)).astype(jnp.int32)
                ce = plsc.load_gather(ex_v, [eidx])
                for k in range(FDIM // 16):
                    ksl = pl.ds(k * 16, 16)
                    rows_v[e, ksl] = rows_v[e, ksl] * ce

            pltpu.sync_copy(rows_v, acc_sh.at[dst_b[b]], add=True)
        return carry

    lax.fori_loop(0, CHUNKS // 2, _pair, 0)

    pltpu.sync_copy(den_v, dpart_hbm.at[wid])
    plsc.subcore_barrier()
    pltpu.sync_copy(acc_sh.at[pl.ds(sid * ROWS_PER_TILE, ROWS_PER_TILE)],
                    acc_hbm.at[cid, pl.ds(sid * ROWS_PER_TILE, ROWS_PER_TILE)])


def _run_sc_edge(h, s, d, ae, src, dst):
    mesh = plsc.VectorSubcoreMesh(core_axis_name="c", subcore_axis_name="s")
    fn = pl.kernel(
        _sc_edge_body,
        out_type=[
            jax.ShapeDtypeStruct((NC, NP, FDIM), jnp.float32),
            jax.ShapeDtypeStruct((NW, NP), jnp.float32),
        ],
        mesh=mesh,
        compiler_params=pltpu.CompilerParams(needs_layout_passes=False),
        scratch_types=[
            pltpu.VMEM((NP,), jnp.float32),
            pltpu.VMEM((NP,), jnp.float32),
            pltpu.VMEM((NP,), jnp.float32),
            pltpu.VMEM((CH,), jnp.int32),
            pltpu.VMEM((CH,), jnp.int32),
            pltpu.VMEM((CH,), jnp.float32),
            pltpu.VMEM((CH, FDIM), jnp.float32),
            pltpu.VMEM((CH,), jnp.int32),
            pltpu.VMEM((CH,), jnp.int32),
            pltpu.VMEM((CH,), jnp.float32),
            pltpu.VMEM((CH, FDIM), jnp.float32),
            pltpu.VMEM((CH,), jnp.float32),
            pltpu.VMEM_SHARED((NP, FDIM), jnp.float32),
            pltpu.SemaphoreType.DMA,
            pltpu.SemaphoreType.DMA,
        ],
    )
    return fn(h, s, d, ae, src, dst)


def kernel(x, edge_index, batch, edge_attr,
           g0_W, g0_asrc, g0_adst, g0_We, g0_ae, g0_b,
           g1_W, g1_asrc, g1_adst, g1_We, g1_ae, g1_b,
           lin0_W, lin0_b, h0_W, h0_b, h1_W, h1_b):
    N = x.shape[0]
    E = edge_index.shape[1]

    av0T = jnp.stack([g0_asrc[0], g0_adst[0]], axis=1)
    av1T = jnp.stack([g1_asrc[0], g1_adst[0]], axis=1)

    AE, csum = _run_ae(edge_attr, g0_We, g0_ae.reshape(-1, 1),
                       g1_We, g1_ae.reshape(-1, 1))

    pad = E_PAD - E
    pad_idx = (jnp.arange(pad, dtype=jnp.int32) % N)
    src_p = jnp.concatenate([edge_index[0], pad_idx])
    dst_p = jnp.concatenate([edge_index[1], pad_idx])
    neg = jnp.full((pad,), -1e30, jnp.float32)
    ae0_p = jnp.concatenate([AE[:, 0], neg])
    ae1_p = jnp.concatenate([AE[:, 1], neg])

    xp = jnp.concatenate([x, jnp.zeros((NP - N, FDIM), jnp.float32)], axis=0)
    batch_p = jnp.concatenate(
        [batch, jnp.full((NP - N,), 16, batch.dtype)]).reshape(-1, 1)

    h0, sd0 = _run_node(xp, g0_W, av0T)
    acc0, dpart0 = _run_sc_edge(h0, sd0[:, 0], sd0[:, 1], ae0_p, src_p, dst_p)
    h1, sd1 = _run_mid(acc0, dpart0, sd0, csum, h0, g0_b.reshape(1, -1),
                       g1_W, av1T)
    acc1, dpart1 = _run_sc_edge(h1, sd1[:, 0], sd1[:, 1], ae1_p, src_p, dst_p)
    out = _run_head(acc1, dpart1, sd1, csum, h1, g1_b.reshape(1, -1),
                    batch_p,
                    lin0_W, lin0_b, h0_W, h0_b, h1_W, h1_b)
    return out


# speedup vs baseline: 26.7133x; 1.3756x over previous
import jax
import jax.numpy as jnp
from jax import lax
from jax.experimental import pallas as pl
from jax.experimental.pallas import tpu as pltpu
from jax.experimental.pallas import tpu_sc as plsc

N_NODES = 10000
NP = 10240
N_EDGES = 320000
FDIM = 128
NC = 2
NS = 16
NW = NC * NS
CH = 64
CHUNKS = 162
EW = CHUNKS * CH
E_PAD = EW * NW
ROWS_PER_TILE = NP // NS
ROW_BLK = 64
NBLK = 2048
NGRID = NP // NBLK


def _ae_body(ea_ref, we0_ref, ae0_ref, we1_ref, ae1_ref,
             out0_ref, out1_ref, csum_ref):
    ve0 = jnp.dot(we0_ref[...], ae0_ref[...], preferred_element_type=jnp.float32)
    ve1 = jnp.dot(we1_ref[...], ae1_ref[...], preferred_element_type=jnp.float32)
    ve = jnp.concatenate([ve0, ve1], axis=1)
    blk = jnp.dot(ea_ref[...], ve, preferred_element_type=jnp.float32)
    out0_ref[...] = blk[:, 0:1]
    out1_ref[...] = blk[:, 1:2]

    @pl.when(pl.program_id(0) == 0)
    def _():
        csum_ref[...] = jnp.zeros_like(csum_ref)

    csum_ref[...] += jnp.sum(blk, axis=0, keepdims=True)


def _run_ae(edge_attr, g0_We, g0_ae_col, g1_We, g1_ae_col):
    E = edge_attr.shape[0]
    BLK = 16000
    grid = (E // BLK,)
    return pl.pallas_call(
        _ae_body,
        grid=grid,
        in_specs=[
            pl.BlockSpec((BLK, edge_attr.shape[1]), lambda i: (i, 0)),
            pl.BlockSpec(g0_We.shape, lambda i: (0, 0)),
            pl.BlockSpec(g0_ae_col.shape, lambda i: (0, 0)),
            pl.BlockSpec(g1_We.shape, lambda i: (0, 0)),
            pl.BlockSpec(g1_ae_col.shape, lambda i: (0, 0)),
        ],
        out_specs=[
            pl.BlockSpec((BLK, 1), lambda i: (i, 0)),
            pl.BlockSpec((BLK, 1), lambda i: (i, 0)),
            pl.BlockSpec((1, 2), lambda i: (0, 0)),
        ],
        out_shape=[
            jax.ShapeDtypeStruct((E, 1), jnp.float32),
            jax.ShapeDtypeStruct((E, 1), jnp.float32),
            jax.ShapeDtypeStruct((1, 2), jnp.float32),
        ],
    )(edge_attr, g0_We, g0_ae_col, g1_We, g1_ae_col)


def _node_body(x_ref, w_ref, avt_ref, h_ref, s_ref, d_ref):
    h = jnp.dot(x_ref[...], w_ref[...], preferred_element_type=jnp.float32)
    h_ref[...] = h
    sd = jnp.dot(h, avt_ref[...], preferred_element_type=jnp.float32)
    s_ref[...] = sd[:, 0:1]
    d_ref[...] = sd[:, 1:2]


def _run_node(x, W, avT):
    return pl.pallas_call(
        _node_body,
        grid=(NGRID,),
        in_specs=[
            pl.BlockSpec((NBLK, FDIM), lambda i: (i, 0)),
            pl.BlockSpec((FDIM, FDIM), lambda i: (0, 0)),
            pl.BlockSpec((FDIM, 2), lambda i: (0, 0)),
        ],
        out_specs=[
            pl.BlockSpec((NBLK, FDIM), lambda i: (i, 0)),
            pl.BlockSpec((NBLK, 1), lambda i: (i, 0)),
            pl.BlockSpec((NBLK, 1), lambda i: (i, 0)),
        ],
        out_shape=[
            jax.ShapeDtypeStruct((NP, FDIM), jnp.float32),
            jax.ShapeDtypeStruct((NP, 1), jnp.float32),
            jax.ShapeDtypeStruct((NP, 1), jnp.float32),
        ],
    )(x, W, avT)


def _mid_body(acc_ref, dpart_ref, s_in, d_in, csum_ref, h_ref, b_ref,
              w1_ref, av1t_ref, h1_ref, s1_ref, d1_ref):
    cl = csum_ref[0, 0] * (1.0 / N_EDGES)
    al = s_in[...] + d_in[...] + cl
    al = jnp.where(al > 0, al, 0.2 * al)
    exl = jnp.exp(al)
    ones = jnp.ones((NW, 1), jnp.float32)
    dsum = lax.dot_general(dpart_ref[...], ones, (((0,), (0,)), ((), ())),
                           preferred_element_type=jnp.float32)
    rden = 1.0 / (dsum + exl + 1e-16)
    h = h_ref[...]
    out = (acc_ref[0] + acc_ref[1] + exl * h) * rden + b_ref[...]
    x1 = jnp.maximum(out, 0.0)
    h1 = jnp.dot(x1, w1_ref[...], preferred_element_type=jnp.float32)
    h1_ref[...] = h1
    sd = jnp.dot(h1, av1t_ref[...], preferred_element_type=jnp.float32)
    s1_ref[...] = sd[:, 0:1]
    d1_ref[...] = sd[:, 1:2]


def _run_mid(acc, dpart, s, d, csum, h, b_row, W1, av1T):
    return pl.pallas_call(
        _mid_body,
        grid=(NGRID,),
        in_specs=[
            pl.BlockSpec((2, NBLK, FDIM), lambda i: (0, i, 0)),
            pl.BlockSpec((NW, NBLK), lambda i: (0, i)),
            pl.BlockSpec((NBLK, 1), lambda i: (i, 0)),
            pl.BlockSpec((NBLK, 1), lambda i: (i, 0)),
            pl.BlockSpec((1, 2), lambda i: (0, 0)),
            pl.BlockSpec((NBLK, FDIM), lambda i: (i, 0)),
            pl.BlockSpec((1, FDIM), lambda i: (0, 0)),
            pl.BlockSpec((FDIM, FDIM), lambda i: (0, 0)),
            pl.BlockSpec((FDIM, 2), lambda i: (0, 0)),
        ],
        out_specs=[
            pl.BlockSpec((NBLK, FDIM), lambda i: (i, 0)),
            pl.BlockSpec((NBLK, 1), lambda i: (i, 0)),
            pl.BlockSpec((NBLK, 1), lambda i: (i, 0)),
        ],
        out_shape=[
            jax.ShapeDtypeStruct((NP, FDIM), jnp.float32),
            jax.ShapeDtypeStruct((NP, 1), jnp.float32),
            jax.ShapeDtypeStruct((NP, 1), jnp.float32),
        ],
    )(acc, dpart, s, d, csum, h, b_row, W1, av1T)


def _head_body(acc_ref, dpart_ref, s_in, d_in, csum_ref, h_ref, b_ref,
               batch_ref, lin0w_ref, lin0b_ref, h0w_ref, h0b_ref,
               h1w_ref, h1b_ref, out_ref, z_scr):
    i = pl.program_id(0)
    cl = csum_ref[0, 1] * (1.0 / N_EDGES)
    al = s_in[...] + d_in[...] + cl
    al = jnp.where(al > 0, al, 0.2 * al)
    exl = jnp.exp(al)
    ones = jnp.ones((NW, 1), jnp.float32)
    dsum = lax.dot_general(dpart_ref[...], ones, (((0,), (0,)), ((), ())),
                           preferred_element_type=jnp.float32)
    rden = 1.0 / (dsum + exl + 1e-16)
    hf = (acc_ref[0] + acc_ref[1] + exl * h_ref[...]) * rden + b_ref[...]

    batch = batch_ref[...]
    gids = lax.broadcasted_iota(jnp.int32, (1, 16), 1)
    lt = (batch < gids).astype(jnp.float32)
    onesn = jnp.ones((NP, 1), jnp.float32)
    counts = lax.dot_general(lt, onesn, (((0,), (0,)), ((), ())),
                             preferred_element_type=jnp.float32
                             ).astype(jnp.int32)
    blk_iota = lax.broadcasted_iota(jnp.int32, (16, NBLK), 1) + i * NBLK
    onehot = (blk_iota == counts).astype(jnp.float32)

    @pl.when(i == 0)
    def _():
        z_scr[...] = jnp.zeros_like(z_scr)

    z_scr[...] += jnp.dot(onehot, hf, preferred_element_type=jnp.float32)

    @pl.when(i == NGRID - 1)
    def _():
        z = z_scr[...]
        z = jnp.maximum(jnp.dot(z, lin0w_ref[...],
                                preferred_element_type=jnp.float32)
                        + lin0b_ref[...], 0.0)
        z = jnp.maximum(jnp.dot(z, h0w_ref[...],
                                preferred_element_type=jnp.float32)
                        + h0b_ref[...], 0.0)
        out_ref[...] = jnp.dot(z, h1w_ref[...],
                               preferred_element_type=jnp.float32) + h1b_ref[...]


def _run_head(acc, dpart, s, d, csum, h, b_row, batch_col,
              lin0_W, lin0_b, h0_W, h0_b, h1_W, h1_b):
    return pl.pallas_call(
        _head_body,
        grid=(NGRID,),
        in_specs=[
            pl.BlockSpec((2, NBLK, FDIM), lambda i: (0, i, 0)),
            pl.BlockSpec((NW, NBLK), lambda i: (0, i)),
            pl.BlockSpec((NBLK, 1), lambda i: (i, 0)),
            pl.BlockSpec((NBLK, 1), lambda i: (i, 0)),
            pl.BlockSpec((1, 2), lambda i: (0, 0)),
            pl.BlockSpec((NBLK, FDIM), lambda i: (i, 0)),
            pl.BlockSpec((1, FDIM), lambda i: (0, 0)),
            pl.BlockSpec((NP, 1), lambda i: (0, 0)),
            pl.BlockSpec((FDIM, FDIM), lambda i: (0, 0)),
            pl.BlockSpec((1, FDIM), lambda i: (0, 0)),
            pl.BlockSpec((FDIM, 64), lambda i: (0, 0)),
            pl.BlockSpec((1, 64), lambda i: (0, 0)),
            pl.BlockSpec((64, 16), lambda i: (0, 0)),
            pl.BlockSpec((1, 16), lambda i: (0, 0)),
        ],
        out_specs=pl.BlockSpec((16, 16), lambda i: (0, 0)),
        out_shape=jax.ShapeDtypeStruct((16, 16), jnp.float32),
        scratch_shapes=[pltpu.VMEM((16, FDIM), jnp.float32)],
    )(acc, dpart, s, d, csum, h, b_row, batch_col,
      lin0_W, lin0_b.reshape(1, -1), h0_W, h0_b.reshape(1, -1),
      h1_W, h1_b.reshape(1, -1))


def _sc_edge_body(h_hbm, s_hbm, d_hbm, ed_hbm,
                  acc_hbm, dpart_hbm,
                  s_v, d_v, den_v,
                  ed0, ed1, ed2, rows0, rows1, ex_v,
                  acc_sh,
                  semi0, semi1, semi2, semg0, semg1, semsc):
    cid = lax.axis_index("c")
    sid = lax.axis_index("s")
    wid = cid * NS + sid

    eds = (ed0, ed1, ed2)
    semis = (semi0, semi1, semi2)
    rows = (rows0, rows1)
    semgs = (semg0, semg1)

    pltpu.sync_copy(s_hbm, s_v)
    pltpu.sync_copy(d_hbm, d_v)

    zero16 = jnp.zeros((16,), jnp.float32)

    @plsc.parallel_loop(0, NP // 16, unroll=4)
    def _(i):
        den_v[pl.ds(i * 16, 16)] = zero16

    @plsc.parallel_loop(0, CH, unroll=4)
    def _(i):
        for k in range(FDIM // 16):
            rows0[i, pl.ds(k * 16, 16)] = zero16

    for j in range(ROWS_PER_TILE // ROW_BLK):
        pltpu.sync_copy(rows0,
                        acc_sh.at[pl.ds(sid * ROWS_PER_TILE + j * ROW_BLK,
                                        ROW_BLK)])
    plsc.subcore_barrier()

    def _stage(i, r):
        pltpu.async_copy(ed_hbm.at[wid, i], eds[r], semis[r])

    def _gather(i, r, b):
        pltpu.async_copy(h_hbm.at[eds[r].at[0]], rows[b], semgs[b])

    _stage(0, 0)
    _stage(1, 1)
    pltpu.make_async_copy(ed_hbm.at[wid, 0], ed0, semi0).wait()
    _gather(0, 0, 0)

    def _chunk(i, r, b, guard):
        ed_r = eds[r]
        rows_b = rows[b]

        rp = (r + 2) % 3
        if guard:
            @pl.when(i > 0)
            def _():
                pltpu.make_async_copy(rows[1 - b],
                                      acc_sh.at[eds[rp].at[1]], semsc).wait()
        else:
            pltpu.make_async_copy(rows[1 - b],
                                  acc_sh.at[eds[rp].at[1]], semsc).wait()

        @pl.when(i + 2 < CHUNKS)
        def _():
            _stage(i + 2, rp)

        @pl.when(i + 1 < CHUNKS)
        def _():
            rn = (r + 1) % 3
            pltpu.make_async_copy(ed_hbm.at[wid, i + 1], eds[rn],
                                  semis[rn]).wait()
            _gather(i + 1, rn, 1 - b)

        pltpu.make_async_copy(h_hbm.at[ed_r.at[0]], rows_b, semgs[b]).wait()

        for g in range(CH // 16):
            sl = pl.ds(g * 16, 16)
            sv = ed_r[0, sl]
            dv = ed_r[1, sl]
            ae = plsc.bitcast(ed_r[2, sl], jnp.float32)
            a = (plsc.load_gather(s_v, [sv]) + plsc.load_gather(d_v, [dv])
                 + ae)
            a = jnp.where(a > 0, a, 0.2 * a)
            ex = jnp.exp(a)
            plsc.addupdate_scatter(den_v, [dv], ex)
            ex_v[sl] = ex

        @plsc.parallel_loop(0, CH, unroll=4)
        def _(e):
            eidx = jnp.broadcast_to(e, (16,)).astype(jnp.int32)
            ce = plsc.load_gather(ex_v, [eidx])
            for k in range(FDIM // 16):
                ksl = pl.ds(k * 16, 16)
                rows_b[e, ksl] = rows_b[e, ksl] * ce

        pltpu.async_copy(rows_b, acc_sh.at[ed_r.at[1]], semsc, add=True)

    def _six(it, carry):
        i0 = it * 6
        for u in range(6):
            _chunk(i0 + u, u % 3, u % 2, u == 0)
        return carry

    lax.fori_loop(0, CHUNKS // 6, _six, 0)

    pltpu.make_async_copy(rows[(CHUNKS - 1) % 2],
                          acc_sh.at[eds[(CHUNKS - 1) % 3].at[1]],
                          semsc).wait()

    pltpu.sync_copy(den_v, dpart_hbm.at[wid])
    plsc.subcore_barrier()
    pltpu.sync_copy(acc_sh.at[pl.ds(sid * ROWS_PER_TILE, ROWS_PER_TILE)],
                    acc_hbm.at[cid, pl.ds(sid * ROWS_PER_TILE, ROWS_PER_TILE)])


def _run_sc_edge(h, s, d, edata):
    mesh = plsc.VectorSubcoreMesh(core_axis_name="c", subcore_axis_name="s")
    fn = pl.kernel(
        _sc_edge_body,
        out_type=[
            jax.ShapeDtypeStruct((NC, NP, FDIM), jnp.float32),
            jax.ShapeDtypeStruct((NW, NP), jnp.float32),
        ],
        mesh=mesh,
        compiler_params=pltpu.CompilerParams(needs_layout_passes=False),
        scratch_types=[
            pltpu.VMEM((NP,), jnp.float32),
            pltpu.VMEM((NP,), jnp.float32),
            pltpu.VMEM((NP,), jnp.float32),
            pltpu.VMEM((3, CH), jnp.int32),
            pltpu.VMEM((3, CH), jnp.int32),
            pltpu.VMEM((3, CH), jnp.int32),
            pltpu.VMEM((CH, FDIM), jnp.float32),
            pltpu.VMEM((CH, FDIM), jnp.float32),
            pltpu.VMEM((CH,), jnp.float32),
            pltpu.VMEM_SHARED((NP, FDIM), jnp.float32),
            pltpu.SemaphoreType.DMA,
            pltpu.SemaphoreType.DMA,
            pltpu.SemaphoreType.DMA,
            pltpu.SemaphoreType.DMA,
            pltpu.SemaphoreType.DMA,
            pltpu.SemaphoreType.DMA,
        ],
    )
    return fn(h, s, d, edata)


def kernel(x, edge_index, batch, edge_attr,
           g0_W, g0_asrc, g0_adst, g0_We, g0_ae, g0_b,
           g1_W, g1_asrc, g1_adst, g1_We, g1_ae, g1_b,
           lin0_W, lin0_b, h0_W, h0_b, h1_W, h1_b):
    N = x.shape[0]
    E = edge_index.shape[1]

    av0T = jnp.stack([g0_asrc[0], g0_adst[0]], axis=1)
    av1T = jnp.stack([g1_asrc[0], g1_adst[0]], axis=1)

    ae0, ae1, csum = _run_ae(edge_attr, g0_We, g0_ae.reshape(-1, 1),
                             g1_We, g1_ae.reshape(-1, 1))

    pad = E_PAD - E
    pad_idx = (jnp.arange(pad, dtype=jnp.int32) % N)
    src_p = jnp.concatenate([edge_index[0], pad_idx]).reshape(NW, CHUNKS, CH)
    dst_p = jnp.concatenate([edge_index[1], pad_idx]).reshape(NW, CHUNKS, CH)
    neg = jnp.full((pad,), -1e30, jnp.float32)

    def _edata(ae_col):
        ae_p = jax.lax.bitcast_convert_type(
            jnp.concatenate([ae_col.reshape(-1), neg]), jnp.int32
        ).reshape(NW, CHUNKS, CH)
        return jnp.stack([src_p, dst_p, ae_p], axis=2)

    edata0 = _edata(ae0)
    edata1 = _edata(ae1)

    xp = jnp.concatenate([x, jnp.zeros((NP - N, FDIM), jnp.float32)], axis=0)
    batch_p = jnp.concatenate(
        [batch, jnp.full((NP - N,), 16, batch.dtype)]).reshape(-1, 1)

    h0, s0, d0 = _run_node(xp, g0_W, av0T)
    acc0, dpart0 = _run_sc_edge(h0, s0.reshape(-1), d0.reshape(-1), edata0)
    h1, s1, d1 = _run_mid(acc0, dpart0, s0, d0, csum, h0,
                          g0_b.reshape(1, -1), g1_W, av1T)
    acc1, dpart1 = _run_sc_edge(h1, s1.reshape(-1), d1.reshape(-1), edata1)
    out = _run_head(acc1, dpart1, s1, d1, csum, h1, g1_b.reshape(1, -1),
                    batch_p, lin0_W, lin0_b, h0_W, h0_b, h1_W, h1_b)
    return out


# speedup vs baseline: 33.1548x; 1.2411x over previous
import jax
import jax.numpy as jnp
from jax import lax
from jax.experimental import pallas as pl
from jax.experimental.pallas import tpu as pltpu
from jax.experimental.pallas import tpu_sc as plsc

N_NODES = 10000
NP = 10240
N_EDGES = 320000
FDIM = 128
NC = 2
NS = 16
NW = NC * NS
CH = 64
CHUNKS = 162
EW = CHUNKS * CH
E_PAD = EW * NW
ROWS_PER_TILE = NP // NS
ROW_BLK = 64
NBLK = 2048
NGRID = NP // NBLK


def _ae_body(ea_ref, we0_ref, ae0_ref, we1_ref, ae1_ref,
             out_ref, csum_ref):
    ve0 = jnp.dot(we0_ref[...], ae0_ref[...], preferred_element_type=jnp.float32)
    ve1 = jnp.dot(we1_ref[...], ae1_ref[...], preferred_element_type=jnp.float32)
    ve_cat = jnp.concatenate([ve0, ve1], axis=1)
    l_row = lax.broadcasted_iota(jnp.int32, (128, 16), 0)
    c_col = lax.broadcasted_iota(jnp.int32, (128, 16), 1)
    T = (l_row % 16 == c_col).astype(jnp.float32)
    vb = jnp.dot(T, ve_cat, preferred_element_type=jnp.float32)
    m_pre = jnp.concatenate(
        [jnp.broadcast_to(vb[:, 0:1], (128, 8)),
         jnp.broadcast_to(vb[:, 1:2], (128, 8))], axis=1)
    mask = (l_row // 16 == c_col % 8).astype(jnp.float32)
    M = m_pre * mask
    out = jnp.dot(ea_ref[...], M, preferred_element_type=jnp.float32)
    out_ref[...] = out

    @pl.when(pl.program_id(0) == 0)
    def _():
        csum_ref[...] = jnp.zeros_like(csum_ref)

    s0 = jnp.sum(out[:, 0:8]).reshape(1, 1)
    s1 = jnp.sum(out[:, 8:16]).reshape(1, 1)
    csum_ref[...] += jnp.concatenate([s0, s1], axis=1)


def _run_ae(ea_packed, g0_We, g0_ae_col, g1_We, g1_ae_col):
    R = ea_packed.shape[0]
    BLK = 8000
    grid = (R // BLK,)
    return pl.pallas_call(
        _ae_body,
        grid=grid,
        in_specs=[
            pl.BlockSpec((BLK, 128), lambda i: (i, 0)),
            pl.BlockSpec(g0_We.shape, lambda i: (0, 0)),
            pl.BlockSpec(g0_ae_col.shape, lambda i: (0, 0)),
            pl.BlockSpec(g1_We.shape, lambda i: (0, 0)),
            pl.BlockSpec(g1_ae_col.shape, lambda i: (0, 0)),
        ],
        out_specs=[
            pl.BlockSpec((BLK, 16), lambda i: (i, 0)),
            pl.BlockSpec((1, 2), lambda i: (0, 0)),
        ],
        out_shape=[
            jax.ShapeDtypeStruct((R, 16), jnp.float32),
            jax.ShapeDtypeStruct((1, 2), jnp.float32),
        ],
    )(ea_packed, g0_We, g0_ae_col, g1_We, g1_ae_col)


def _node_body(x_ref, w_ref, avt_ref, h_ref, s_ref, d_ref):
    h = jnp.dot(x_ref[...], w_ref[...], preferred_element_type=jnp.float32)
    h_ref[...] = h
    sd = jnp.dot(h, avt_ref[...], preferred_element_type=jnp.float32)
    s_ref[...] = sd[:, 0:1]
    d_ref[...] = sd[:, 1:2]


def _run_node(x, W, avT):
    return pl.pallas_call(
        _node_body,
        grid=(NGRID,),
        in_specs=[
            pl.BlockSpec((NBLK, FDIM), lambda i: (i, 0)),
            pl.BlockSpec((FDIM, FDIM), lambda i: (0, 0)),
            pl.BlockSpec((FDIM, 2), lambda i: (0, 0)),
        ],
        out_specs=[
            pl.BlockSpec((NBLK, FDIM), lambda i: (i, 0)),
            pl.BlockSpec((NBLK, 1), lambda i: (i, 0)),
            pl.BlockSpec((NBLK, 1), lambda i: (i, 0)),
        ],
        out_shape=[
            jax.ShapeDtypeStruct((NP, FDIM), jnp.float32),
            jax.ShapeDtypeStruct((NP, 1), jnp.float32),
            jax.ShapeDtypeStruct((NP, 1), jnp.float32),
        ],
    )(x, W, avT)


def _mid_body(acc_ref, dpart_ref, s_in, d_in, csum_ref, h_ref, b_ref,
              w1_ref, av1t_ref, h1_ref, s1_ref, d1_ref):
    cl = csum_ref[0, 0] * (1.0 / N_EDGES)
    al = s_in[...] + d_in[...] + cl
    al = jnp.where(al > 0, al, 0.2 * al)
    exl = jnp.exp(al)
    ones = jnp.ones((NW, 1), jnp.float32)
    dsum = lax.dot_general(dpart_ref[...], ones, (((0,), (0,)), ((), ())),
                           preferred_element_type=jnp.float32)
    rden = 1.0 / (dsum + exl + 1e-16)
    h = h_ref[...]
    out = (acc_ref[0] + acc_ref[1] + exl * h) * rden + b_ref[...]
    x1 = jnp.maximum(out, 0.0)
    h1 = jnp.dot(x1, w1_ref[...], preferred_element_type=jnp.float32)
    h1_ref[...] = h1
    sd = jnp.dot(h1, av1t_ref[...], preferred_element_type=jnp.float32)
    s1_ref[...] = sd[:, 0:1]
    d1_ref[...] = sd[:, 1:2]


def _run_mid(acc, dpart, s, d, csum, h, b_row, W1, av1T):
    return pl.pallas_call(
        _mid_body,
        grid=(NGRID,),
        in_specs=[
            pl.BlockSpec((2, NBLK, FDIM), lambda i: (0, i, 0)),
            pl.BlockSpec((NW, NBLK), lambda i: (0, i)),
            pl.BlockSpec((NBLK, 1), lambda i: (i, 0)),
            pl.BlockSpec((NBLK, 1), lambda i: (i, 0)),
            pl.BlockSpec((1, 2), lambda i: (0, 0)),
            pl.BlockSpec((NBLK, FDIM), lambda i: (i, 0)),
            pl.BlockSpec((1, FDIM), lambda i: (0, 0)),
            pl.BlockSpec((FDIM, FDIM), lambda i: (0, 0)),
            pl.BlockSpec((FDIM, 2), lambda i: (0, 0)),
        ],
        out_specs=[
            pl.BlockSpec((NBLK, FDIM), lambda i: (i, 0)),
            pl.BlockSpec((NBLK, 1), lambda i: (i, 0)),
            pl.BlockSpec((NBLK, 1), lambda i: (i, 0)),
        ],
        out_shape=[
            jax.ShapeDtypeStruct((NP, FDIM), jnp.float32),
            jax.ShapeDtypeStruct((NP, 1), jnp.float32),
            jax.ShapeDtypeStruct((NP, 1), jnp.float32),
        ],
    )(acc, dpart, s, d, csum, h, b_row, W1, av1T)


def _head_body(acc_ref, dpart_ref, s_in, d_in, csum_ref, h_ref, b_ref,
               batch_ref, lin0w_ref, lin0b_ref, h0w_ref, h0b_ref,
               h1w_ref, h1b_ref, out_ref, z_scr):
    i = pl.program_id(0)
    cl = csum_ref[0, 1] * (1.0 / N_EDGES)
    al = s_in[...] + d_in[...] + cl
    al = jnp.where(al > 0, al, 0.2 * al)
    exl = jnp.exp(al)
    ones = jnp.ones((NW, 1), jnp.float32)
    dsum = lax.dot_general(dpart_ref[...], ones, (((0,), (0,)), ((), ())),
                           preferred_element_type=jnp.float32)
    rden = 1.0 / (dsum + exl + 1e-16)
    hf = (acc_ref[0] + acc_ref[1] + exl * h_ref[...]) * rden + b_ref[...]

    batch = batch_ref[...]
    gids = lax.broadcasted_iota(jnp.int32, (1, 16), 1)
    lt = (batch < gids).astype(jnp.float32)
    onesn = jnp.ones((NP, 1), jnp.float32)
    counts = lax.dot_general(lt, onesn, (((0,), (0,)), ((), ())),
                             preferred_element_type=jnp.float32
                             ).astype(jnp.int32)
    blk_iota = lax.broadcasted_iota(jnp.int32, (16, NBLK), 1) + i * NBLK
    onehot = (blk_iota == counts).astype(jnp.float32)

    @pl.when(i == 0)
    def _():
        z_scr[...] = jnp.zeros_like(z_scr)

    z_scr[...] += jnp.dot(onehot, hf, preferred_element_type=jnp.float32)

    @pl.when(i == NGRID - 1)
    def _():
        z = z_scr[...]
        z = jnp.maximum(jnp.dot(z, lin0w_ref[...],
                                preferred_element_type=jnp.float32)
                        + lin0b_ref[...], 0.0)
        z = jnp.maximum(jnp.dot(z, h0w_ref[...],
                                preferred_element_type=jnp.float32)
                        + h0b_ref[...], 0.0)
        out_ref[...] = jnp.dot(z, h1w_ref[...],
                               preferred_element_type=jnp.float32) + h1b_ref[...]


def _run_head(acc, dpart, s, d, csum, h, b_row, batch_col,
              lin0_W, lin0_b, h0_W, h0_b, h1_W, h1_b):
    return pl.pallas_call(
        _head_body,
        grid=(NGRID,),
        in_specs=[
            pl.BlockSpec((2, NBLK, FDIM), lambda i: (0, i, 0)),
            pl.BlockSpec((NW, NBLK), lambda i: (0, i)),
            pl.BlockSpec((NBLK, 1), lambda i: (i, 0)),
            pl.BlockSpec((NBLK, 1), lambda i: (i, 0)),
            pl.BlockSpec((1, 2), lambda i: (0, 0)),
            pl.BlockSpec((NBLK, FDIM), lambda i: (i, 0)),
            pl.BlockSpec((1, FDIM), lambda i: (0, 0)),
            pl.BlockSpec((NP, 1), lambda i: (0, 0)),
            pl.BlockSpec((FDIM, FDIM), lambda i: (0, 0)),
            pl.BlockSpec((1, FDIM), lambda i: (0, 0)),
            pl.BlockSpec((FDIM, 64), lambda i: (0, 0)),
            pl.BlockSpec((1, 64), lambda i: (0, 0)),
            pl.BlockSpec((64, 16), lambda i: (0, 0)),
            pl.BlockSpec((1, 16), lambda i: (0, 0)),
        ],
        out_specs=pl.BlockSpec((16, 16), lambda i: (0, 0)),
        out_shape=jax.ShapeDtypeStruct((16, 16), jnp.float32),
        scratch_shapes=[pltpu.VMEM((16, FDIM), jnp.float32)],
    )(acc, dpart, s, d, csum, h, b_row, batch_col,
      lin0_W, lin0_b.reshape(1, -1), h0_W, h0_b.reshape(1, -1),
      h1_W, h1_b.reshape(1, -1))


def _sc_edge_body(h_hbm, s_hbm, d_hbm, ed_hbm,
                  acc_hbm, dpart_hbm,
                  s_v, d_v, den_v,
                  ed0, ed1, ed2, rows0, rows1, ex_v,
                  acc_sh,
                  semi0, semi1, semi2, semg0, semg1, semsc):
    cid = lax.axis_index("c")
    sid = lax.axis_index("s")
    wid = cid * NS + sid

    eds = (ed0, ed1, ed2)
    semis = (semi0, semi1, semi2)
    rows = (rows0, rows1)
    semgs = (semg0, semg1)

    pltpu.sync_copy(s_hbm, s_v)
    pltpu.sync_copy(d_hbm, d_v)

    zero16 = jnp.zeros((16,), jnp.float32)

    @plsc.parallel_loop(0, NP // 16, unroll=4)
    def _(i):
        den_v[pl.ds(i * 16, 16)] = zero16

    @plsc.parallel_loop(0, CH, unroll=4)
    def _(i):
        for k in range(FDIM // 16):
            rows0[i, pl.ds(k * 16, 16)] = zero16

    for j in range(ROWS_PER_TILE // ROW_BLK):
        pltpu.sync_copy(rows0,
                        acc_sh.at[pl.ds(sid * ROWS_PER_TILE + j * ROW_BLK,
                                        ROW_BLK)])
    plsc.subcore_barrier()

    def _stage(i, r):
        pltpu.async_copy(ed_hbm.at[wid, i], eds[r], semis[r])

    def _gather(i, r, b):
        pltpu.async_copy(h_hbm.at[eds[r].at[0]], rows[b], semgs[b])

    _stage(0, 0)
    _stage(1, 1)
    pltpu.make_async_copy(ed_hbm.at[wid, 0], ed0, semi0).wait()
    _gather(0, 0, 0)

    def _chunk(i, r, b, guard):
        ed_r = eds[r]
        rows_b = rows[b]

        rp = (r + 2) % 3
        if guard:
            @pl.when(i > 0)
            def _():
                pltpu.make_async_copy(rows[1 - b],
                                      acc_sh.at[eds[rp].at[1]], semsc).wait()
        else:
            pltpu.make_async_copy(rows[1 - b],
                                  acc_sh.at[eds[rp].at[1]], semsc).wait()

        @pl.when(i + 2 < CHUNKS)
        def _():
            _stage(i + 2, rp)

        @pl.when(i + 1 < CHUNKS)
        def _():
            rn = (r + 1) % 3
            pltpu.make_async_copy(ed_hbm.at[wid, i + 1], eds[rn],
                                  semis[rn]).wait()
            _gather(i + 1, rn, 1 - b)

        pltpu.make_async_copy(h_hbm.at[ed_r.at[0]], rows_b, semgs[b]).wait()

        for g in range(CH // 16):
            sl = pl.ds(g * 16, 16)
            sv = ed_r[0, sl]
            dv = ed_r[1, sl]
            ae = plsc.bitcast(ed_r[2, sl], jnp.float32)
            a = (plsc.load_gather(s_v, [sv]) + plsc.load_gather(d_v, [dv])
                 + ae)
            a = jnp.where(a > 0, a, 0.2 * a)
            ex = jnp.exp(a)
            plsc.addupdate_scatter(den_v, [dv], ex)
            ex_v[sl] = ex

        @plsc.parallel_loop(0, CH, unroll=4)
        def _(e):
            eidx = jnp.broadcast_to(e, (16,)).astype(jnp.int32)
            ce = plsc.load_gather(ex_v, [eidx])
            for k in range(FDIM // 16):
                ksl = pl.ds(k * 16, 16)
                rows_b[e, ksl] = rows_b[e, ksl] * ce

        pltpu.async_copy(rows_b, acc_sh.at[ed_r.at[1]], semsc, add=True)

    def _six(it, carry):
        i0 = it * 6
        for u in range(6):
            _chunk(i0 + u, u % 3, u % 2, u == 0)
        return carry

    lax.fori_loop(0, CHUNKS // 6, _six, 0)

    pltpu.make_async_copy(rows[(CHUNKS - 1) % 2],
                          acc_sh.at[eds[(CHUNKS - 1) % 3].at[1]],
                          semsc).wait()

    pltpu.sync_copy(den_v, dpart_hbm.at[wid])
    plsc.subcore_barrier()
    pltpu.sync_copy(acc_sh.at[pl.ds(sid * ROWS_PER_TILE, ROWS_PER_TILE)],
                    acc_hbm.at[cid, pl.ds(sid * ROWS_PER_TILE, ROWS_PER_TILE)])


def _run_sc_edge(h, s, d, edata):
    mesh = plsc.VectorSubcoreMesh(core_axis_name="c", subcore_axis_name="s")
    fn = pl.kernel(
        _sc_edge_body,
        out_type=[
            jax.ShapeDtypeStruct((NC, NP, FDIM), jnp.float32),
            jax.ShapeDtypeStruct((NW, NP), jnp.float32),
        ],
        mesh=mesh,
        compiler_params=pltpu.CompilerParams(needs_layout_passes=False),
        scratch_types=[
            pltpu.VMEM((NP,), jnp.float32),
            pltpu.VMEM((NP,), jnp.float32),
            pltpu.VMEM((NP,), jnp.float32),
            pltpu.VMEM((3, CH), jnp.int32),
            pltpu.VMEM((3, CH), jnp.int32),
            pltpu.VMEM((3, CH), jnp.int32),
            pltpu.VMEM((CH, FDIM), jnp.float32),
            pltpu.VMEM((CH, FDIM), jnp.float32),
            pltpu.VMEM((CH,), jnp.float32),
            pltpu.VMEM_SHARED((NP, FDIM), jnp.float32),
            pltpu.SemaphoreType.DMA,
            pltpu.SemaphoreType.DMA,
            pltpu.SemaphoreType.DMA,
            pltpu.SemaphoreType.DMA,
            pltpu.SemaphoreType.DMA,
            pltpu.SemaphoreType.DMA,
        ],
    )
    return fn(h, s, d, edata)


def kernel(x, edge_index, batch, edge_attr,
           g0_W, g0_asrc, g0_adst, g0_We, g0_ae, g0_b,
           g1_W, g1_asrc, g1_adst, g1_We, g1_ae, g1_b,
           lin0_W, lin0_b, h0_W, h0_b, h1_W, h1_b):
    N = x.shape[0]
    E = edge_index.shape[1]

    av0T = jnp.stack([g0_asrc[0], g0_adst[0]], axis=1)
    av1T = jnp.stack([g1_asrc[0], g1_adst[0]], axis=1)

    ea_packed = edge_attr.reshape(E // 8, 8 * edge_attr.shape[1])
    aep, csum = _run_ae(ea_packed, g0_We, g0_ae.reshape(-1, 1),
                        g1_We, g1_ae.reshape(-1, 1))
    ae0 = aep[:, 0:8]
    ae1 = aep[:, 8:16]

    pad = E_PAD - E
    pad_idx = (jnp.arange(pad, dtype=jnp.int32) % N)
    src_p = jnp.concatenate([edge_index[0], pad_idx]).reshape(NW, CHUNKS, CH)
    dst_p = jnp.concatenate([edge_index[1], pad_idx]).reshape(NW, CHUNKS, CH)
    neg = jnp.full((pad,), -1e30, jnp.float32)

    def _edata(ae_col):
        ae_p = jax.lax.bitcast_convert_type(
            jnp.concatenate([ae_col.reshape(-1), neg]), jnp.int32
        ).reshape(NW, CHUNKS, CH)
        return jnp.stack([src_p, dst_p, ae_p], axis=2)

    edata0 = _edata(ae0)
    edata1 = _edata(ae1)

    xp = jnp.concatenate([x, jnp.zeros((NP - N, FDIM), jnp.float32)], axis=0)
    batch_p = jnp.concatenate(
        [batch, jnp.full((NP - N,), 16, batch.dtype)]).reshape(-1, 1)

    h0, s0, d0 = _run_node(xp, g0_W, av0T)
    acc0, dpart0 = _run_sc_edge(h0, s0.reshape(-1), d0.reshape(-1), edata0)
    h1, s1, d1 = _run_mid(acc0, dpart0, s0, d0, csum, h0,
                          g0_b.reshape(1, -1), g1_W, av1T)
    acc1, dpart1 = _run_sc_edge(h1, s1.reshape(-1), d1.reshape(-1), edata1)
    out = _run_head(acc1, dpart1, s1, d1, csum, h1, g1_b.reshape(1, -1),
                    batch_p, lin0_W, lin0_b, h0_W, h0_b, h1_W, h1_b)
    return out


# speedup vs baseline: 34.9334x; 1.0536x over previous
import jax
import jax.numpy as jnp
from jax import lax
from jax.experimental import pallas as pl
from jax.experimental.pallas import tpu as pltpu
from jax.experimental.pallas import tpu_sc as plsc

N_NODES = 10000
NP = 10240
N_EDGES = 320000
FDIM = 128
NC = 2
NS = 16
NW = NC * NS
CH = 64
CHUNKS = 162
EW = CHUNKS * CH
E_PAD = EW * NW
ROWS_PER_TILE = NP // NS
ROW_BLK = 64
NBLK = 2048
NGRID = NP // NBLK


def _ae_body(ea_ref, we0_ref, ae0_ref, we1_ref, ae1_ref,
             out_ref, csum_ref):
    ve0 = jnp.dot(we0_ref[...], ae0_ref[...], preferred_element_type=jnp.float32)
    ve1 = jnp.dot(we1_ref[...], ae1_ref[...], preferred_element_type=jnp.float32)
    ve_cat = jnp.concatenate([ve0, ve1], axis=1)
    l_row = lax.broadcasted_iota(jnp.int32, (128, 16), 0)
    c_col = lax.broadcasted_iota(jnp.int32, (128, 16), 1)
    T = (l_row % 16 == c_col).astype(jnp.float32)
    vb = jnp.dot(T, ve_cat, preferred_element_type=jnp.float32)
    m_pre = jnp.concatenate(
        [jnp.broadcast_to(vb[:, 0:1], (128, 8)),
         jnp.broadcast_to(vb[:, 1:2], (128, 8))], axis=1)
    mask = (l_row // 16 == c_col % 8).astype(jnp.float32)
    M = m_pre * mask
    out = jnp.dot(ea_ref[...], M, preferred_element_type=jnp.float32)
    out_ref[...] = out

    @pl.when(pl.program_id(0) == 0)
    def _():
        csum_ref[...] = jnp.zeros_like(csum_ref)

    s0 = jnp.sum(out[:, 0:8]).reshape(1, 1)
    s1 = jnp.sum(out[:, 8:16]).reshape(1, 1)
    csum_ref[...] += jnp.concatenate([s0, s1], axis=1)


def _run_ae(ea_packed, g0_We, g0_ae_col, g1_We, g1_ae_col):
    R = ea_packed.shape[0]
    BLK = 8000
    grid = (R // BLK,)
    return pl.pallas_call(
        _ae_body,
        grid=grid,
        in_specs=[
            pl.BlockSpec((BLK, 128), lambda i: (i, 0)),
            pl.BlockSpec(g0_We.shape, lambda i: (0, 0)),
            pl.BlockSpec(g0_ae_col.shape, lambda i: (0, 0)),
            pl.BlockSpec(g1_We.shape, lambda i: (0, 0)),
            pl.BlockSpec(g1_ae_col.shape, lambda i: (0, 0)),
        ],
        out_specs=[
            pl.BlockSpec((BLK, 16), lambda i: (i, 0)),
            pl.BlockSpec((1, 2), lambda i: (0, 0)),
        ],
        out_shape=[
            jax.ShapeDtypeStruct((R, 16), jnp.float32),
            jax.ShapeDtypeStruct((1, 2), jnp.float32),
        ],
    )(ea_packed, g0_We, g0_ae_col, g1_We, g1_ae_col)


def _node_body(x_ref, w_ref, avt_ref, h_ref, s_ref, d_ref):
    h = jnp.dot(x_ref[...], w_ref[...], preferred_element_type=jnp.float32)
    h_ref[...] = h
    sd = jnp.dot(h, avt_ref[...], preferred_element_type=jnp.float32)
    s_ref[...] = sd[:, 0:1]
    d_ref[...] = sd[:, 1:2]


def _run_node(x, W, avT):
    return pl.pallas_call(
        _node_body,
        grid=(NGRID,),
        in_specs=[
            pl.BlockSpec((NBLK, FDIM), lambda i: (i, 0)),
            pl.BlockSpec((FDIM, FDIM), lambda i: (0, 0)),
            pl.BlockSpec((FDIM, 2), lambda i: (0, 0)),
        ],
        out_specs=[
            pl.BlockSpec((NBLK, FDIM), lambda i: (i, 0)),
            pl.BlockSpec((NBLK, 1), lambda i: (i, 0)),
            pl.BlockSpec((NBLK, 1), lambda i: (i, 0)),
        ],
        out_shape=[
            jax.ShapeDtypeStruct((NP, FDIM), jnp.float32),
            jax.ShapeDtypeStruct((NP, 1), jnp.float32),
            jax.ShapeDtypeStruct((NP, 1), jnp.float32),
        ],
    )(x, W, avT)


def _mid_body(acc_ref, dpart_ref, s_in, d_in, csum_ref, h_ref, b_ref,
              w1_ref, av1t_ref, h1_ref, s1_ref, d1_ref):
    cl = csum_ref[0, 0] * (1.0 / N_EDGES)
    al = s_in[...] + d_in[...] + cl
    al = jnp.where(al > 0, al, 0.2 * al)
    exl = jnp.exp(al)
    ones = jnp.ones((NW, 1), jnp.float32)
    dsum = lax.dot_general(dpart_ref[...], ones, (((0,), (0,)), ((), ())),
                           preferred_element_type=jnp.float32)
    rden = 1.0 / (dsum + exl + 1e-16)
    h = h_ref[...]
    out = (acc_ref[0] + acc_ref[1] + exl * h) * rden + b_ref[...]
    x1 = jnp.maximum(out, 0.0)
    h1 = jnp.dot(x1, w1_ref[...], preferred_element_type=jnp.float32)
    h1_ref[...] = h1
    sd = jnp.dot(h1, av1t_ref[...], preferred_element_type=jnp.float32)
    s1_ref[...] = sd[:, 0:1]
    d1_ref[...] = sd[:, 1:2]


def _run_mid(acc, dpart, s, d, csum, h, b_row, W1, av1T):
    return pl.pallas_call(
        _mid_body,
        grid=(NGRID,),
        in_specs=[
            pl.BlockSpec((2, NBLK, FDIM), lambda i: (0, i, 0)),
            pl.BlockSpec((NW, NBLK), lambda i: (0, i)),
            pl.BlockSpec((NBLK, 1), lambda i: (i, 0)),
            pl.BlockSpec((NBLK, 1), lambda i: (i, 0)),
            pl.BlockSpec((1, 2), lambda i: (0, 0)),
            pl.BlockSpec((NBLK, FDIM), lambda i: (i, 0)),
            pl.BlockSpec((1, FDIM), lambda i: (0, 0)),
            pl.BlockSpec((FDIM, FDIM), lambda i: (0, 0)),
            pl.BlockSpec((FDIM, 2), lambda i: (0, 0)),
        ],
        out_specs=[
            pl.BlockSpec((NBLK, FDIM), lambda i: (i, 0)),
            pl.BlockSpec((NBLK, 1), lambda i: (i, 0)),
            pl.BlockSpec((NBLK, 1), lambda i: (i, 0)),
        ],
        out_shape=[
            jax.ShapeDtypeStruct((NP, FDIM), jnp.float32),
            jax.ShapeDtypeStruct((NP, 1), jnp.float32),
            jax.ShapeDtypeStruct((NP, 1), jnp.float32),
        ],
    )(acc, dpart, s, d, csum, h, b_row, W1, av1T)


def _head_body(acc_ref, dpart_ref, s_in, d_in, csum_ref, h_ref, b_ref,
               batch_ref, lin0w_ref, lin0b_ref, h0w_ref, h0b_ref,
               h1w_ref, h1b_ref, out_ref, z_scr):
    i = pl.program_id(0)
    cl = csum_ref[0, 1] * (1.0 / N_EDGES)
    al = s_in[...] + d_in[...] + cl
    al = jnp.where(al > 0, al, 0.2 * al)
    exl = jnp.exp(al)
    ones = jnp.ones((NW, 1), jnp.float32)
    dsum = lax.dot_general(dpart_ref[...], ones, (((0,), (0,)), ((), ())),
                           preferred_element_type=jnp.float32)
    rden = 1.0 / (dsum + exl + 1e-16)
    hf = (acc_ref[0] + acc_ref[1] + exl * h_ref[...]) * rden + b_ref[...]

    batch = batch_ref[...]
    gids = lax.broadcasted_iota(jnp.int32, (1, 16), 1)
    lt = (batch < gids).astype(jnp.float32)
    onesn = jnp.ones((NP, 1), jnp.float32)
    counts = lax.dot_general(lt, onesn, (((0,), (0,)), ((), ())),
                             preferred_element_type=jnp.float32
                             ).astype(jnp.int32)
    blk_iota = lax.broadcasted_iota(jnp.int32, (16, NBLK), 1) + i * NBLK
    onehot = (blk_iota == counts).astype(jnp.float32)

    @pl.when(i == 0)
    def _():
        z_scr[...] = jnp.zeros_like(z_scr)

    z_scr[...] += jnp.dot(onehot, hf, preferred_element_type=jnp.float32)

    @pl.when(i == NGRID - 1)
    def _():
        z = z_scr[...]
        z = jnp.maximum(jnp.dot(z, lin0w_ref[...],
                                preferred_element_type=jnp.float32)
                        + lin0b_ref[...], 0.0)
        z = jnp.maximum(jnp.dot(z, h0w_ref[...],
                                preferred_element_type=jnp.float32)
                        + h0b_ref[...], 0.0)
        out_ref[...] = jnp.dot(z, h1w_ref[...],
                               preferred_element_type=jnp.float32) + h1b_ref[...]


def _run_head(acc, dpart, s, d, csum, h, b_row, batch_col,
              lin0_W, lin0_b, h0_W, h0_b, h1_W, h1_b):
    return pl.pallas_call(
        _head_body,
        grid=(NGRID,),
        in_specs=[
            pl.BlockSpec((2, NBLK, FDIM), lambda i: (0, i, 0)),
            pl.BlockSpec((NW, NBLK), lambda i: (0, i)),
            pl.BlockSpec((NBLK, 1), lambda i: (i, 0)),
            pl.BlockSpec((NBLK, 1), lambda i: (i, 0)),
            pl.BlockSpec((1, 2), lambda i: (0, 0)),
            pl.BlockSpec((NBLK, FDIM), lambda i: (i, 0)),
            pl.BlockSpec((1, FDIM), lambda i: (0, 0)),
            pl.BlockSpec((NP, 1), lambda i: (0, 0)),
            pl.BlockSpec((FDIM, FDIM), lambda i: (0, 0)),
            pl.BlockSpec((1, FDIM), lambda i: (0, 0)),
            pl.BlockSpec((FDIM, 64), lambda i: (0, 0)),
            pl.BlockSpec((1, 64), lambda i: (0, 0)),
            pl.BlockSpec((64, 16), lambda i: (0, 0)),
            pl.BlockSpec((1, 16), lambda i: (0, 0)),
        ],
        out_specs=pl.BlockSpec((16, 16), lambda i: (0, 0)),
        out_shape=jax.ShapeDtypeStruct((16, 16), jnp.float32),
        scratch_shapes=[pltpu.VMEM((16, FDIM), jnp.float32)],
    )(acc, dpart, s, d, csum, h, b_row, batch_col,
      lin0_W, lin0_b.reshape(1, -1), h0_W, h0_b.reshape(1, -1),
      h1_W, h1_b.reshape(1, -1))


def _sc_edge_body(h_hbm, s_hbm, d_hbm, src_hbm, dst_hbm, ae_hbm,
                  acc_hbm, dpart_hbm,
                  s_v, d_v, den_v,
                  sv0, sv1, sv2, dv0, dv1, dv2, av0, av1, av2,
                  rows0, rows1, ex_v,
                  acc_sh,
                  semi0, semi1, semi2, semg0, semg1, semsc):
    cid = lax.axis_index("c")
    sid = lax.axis_index("s")
    wid = cid * NS + sid
    wbase = wid * EW

    svs = (sv0, sv1, sv2)
    dvs = (dv0, dv1, dv2)
    avs = (av0, av1, av2)
    semis = (semi0, semi1, semi2)
    rows = (rows0, rows1)
    semgs = (semg0, semg1)

    pltpu.sync_copy(s_hbm, s_v)
    pltpu.sync_copy(d_hbm, d_v)

    zero16 = jnp.zeros((16,), jnp.float32)

    @plsc.parallel_loop(0, NP // 16, unroll=4)
    def _(i):
        den_v[pl.ds(i * 16, 16)] = zero16

    @plsc.parallel_loop(0, CH, unroll=4)
    def _(i):
        for k in range(FDIM // 16):
            rows0[i, pl.ds(k * 16, 16)] = zero16

    for j in range(ROWS_PER_TILE // ROW_BLK):
        pltpu.sync_copy(rows0,
                        acc_sh.at[pl.ds(sid * ROWS_PER_TILE + j * ROW_BLK,
                                        ROW_BLK)])
    plsc.subcore_barrier()

    def _stage(i, r):
        base = wbase + i * CH
        pltpu.async_copy(src_hbm.at[pl.ds(base, CH)], svs[r], semis[r])
        pltpu.async_copy(dst_hbm.at[pl.ds(base, CH)], dvs[r], semis[r])
        pltpu.async_copy(ae_hbm.at[pl.ds(base, CH)], avs[r], semis[r])

    def _stage_wait(i, r):
        base = wbase + i * CH
        pltpu.make_async_copy(src_hbm.at[pl.ds(base, CH)], svs[r],
                              semis[r]).wait()
        pltpu.make_async_copy(dst_hbm.at[pl.ds(base, CH)], dvs[r],
                              semis[r]).wait()
        pltpu.make_async_copy(ae_hbm.at[pl.ds(base, CH)], avs[r],
                              semis[r]).wait()

    def _gather(r, b):
        pltpu.async_copy(h_hbm.at[svs[r]], rows[b], semgs[b])

    _stage(0, 0)
    _stage(1, 1)
    _stage_wait(0, 0)
    _gather(0, 0)

    def _chunk(i, r, b, guard):
        rows_b = rows[b]
        rp = (r + 2) % 3

        if guard:
            @pl.when(i > 0)
            def _():
                pltpu.make_async_copy(rows[1 - b],
                                      acc_sh.at[dvs[rp]], semsc).wait()
        else:
            pltpu.make_async_copy(rows[1 - b],
                                  acc_sh.at[dvs[rp]], semsc).wait()

        @pl.when(i + 2 < CHUNKS)
        def _():
            _stage(i + 2, rp)

        @pl.when(i + 1 < CHUNKS)
        def _():
            rn = (r + 1) % 3
            _stage_wait(i + 1, rn)
            _gather(rn, 1 - b)

        pltpu.make_async_copy(h_hbm.at[svs[r]], rows_b, semgs[b]).wait()

        for g in range(CH // 16):
            sl = pl.ds(g * 16, 16)
            sv = svs[r][sl]
            dv = dvs[r][sl]
            a = (plsc.load_gather(s_v, [sv]) + plsc.load_gather(d_v, [dv])
                 + avs[r][sl])
            a = jnp.where(a > 0, a, 0.2 * a)
            ex = jnp.exp(a)
            plsc.addupdate_scatter(den_v, [dv], ex)
            ex_v[sl] = ex

        @plsc.parallel_loop(0, CH, unroll=8)
        def _(e):
            eidx = jnp.broadcast_to(e, (16,)).astype(jnp.int32)
            ce = plsc.load_gather(ex_v, [eidx])
            for k in range(FDIM // 16):
                ksl = pl.ds(k * 16, 16)
                rows_b[e, ksl] = rows_b[e, ksl] * ce

        pltpu.async_copy(rows_b, acc_sh.at[dvs[r]], semsc, add=True)

    def _six(it, carry):
        i0 = it * 6
        for u in range(6):
            _chunk(i0 + u, u % 3, u % 2, u == 0)
        return carry

    lax.fori_loop(0, CHUNKS // 6, _six, 0)

    pltpu.make_async_copy(rows[(CHUNKS - 1) % 2],
                          acc_sh.at[dvs[(CHUNKS - 1) % 3]],
                          semsc).wait()

    pltpu.sync_copy(den_v, dpart_hbm.at[wid])
    plsc.subcore_barrier()
    pltpu.sync_copy(acc_sh.at[pl.ds(sid * ROWS_PER_TILE, ROWS_PER_TILE)],
                    acc_hbm.at[cid, pl.ds(sid * ROWS_PER_TILE, ROWS_PER_TILE)])


def _run_sc_edge(h, s, d, src_p, dst_p, ae_p):
    mesh = plsc.VectorSubcoreMesh(core_axis_name="c", subcore_axis_name="s")
    fn = pl.kernel(
        _sc_edge_body,
        out_type=[
            jax.ShapeDtypeStruct((NC, NP, FDIM), jnp.float32),
            jax.ShapeDtypeStruct((NW, NP), jnp.float32),
        ],
        mesh=mesh,
        compiler_params=pltpu.CompilerParams(needs_layout_passes=False),
        scratch_types=[
            pltpu.VMEM((NP,), jnp.float32),
            pltpu.VMEM((NP,), jnp.float32),
            pltpu.VMEM((NP,), jnp.float32),
            pltpu.VMEM((CH,), jnp.int32),
            pltpu.VMEM((CH,), jnp.int32),
            pltpu.VMEM((CH,), jnp.int32),
            pltpu.VMEM((CH,), jnp.int32),
            pltpu.VMEM((CH,), jnp.int32),
            pltpu.VMEM((CH,), jnp.int32),
            pltpu.VMEM((CH,), jnp.float32),
            pltpu.VMEM((CH,), jnp.float32),
            pltpu.VMEM((CH,), jnp.float32),
            pltpu.VMEM((CH, FDIM), jnp.float32),
            pltpu.VMEM((CH, FDIM), jnp.float32),
            pltpu.VMEM((CH,), jnp.float32),
            pltpu.VMEM_SHARED((NP, FDIM), jnp.float32),
            pltpu.SemaphoreType.DMA,
            pltpu.SemaphoreType.DMA,
            pltpu.SemaphoreType.DMA,
            pltpu.SemaphoreType.DMA,
            pltpu.SemaphoreType.DMA,
            pltpu.SemaphoreType.DMA,
        ],
    )
    return fn(h, s, d, src_p, dst_p, ae_p)


def kernel(x, edge_index, batch, edge_attr,
           g0_W, g0_asrc, g0_adst, g0_We, g0_ae, g0_b,
           g1_W, g1_asrc, g1_adst, g1_We, g1_ae, g1_b,
           lin0_W, lin0_b, h0_W, h0_b, h1_W, h1_b):
    N = x.shape[0]
    E = edge_index.shape[1]

    av0T = jnp.stack([g0_asrc[0], g0_adst[0]], axis=1)
    av1T = jnp.stack([g1_asrc[0], g1_adst[0]], axis=1)

    ea_packed = edge_attr.reshape(E // 8, 8 * edge_attr.shape[1])
    aep, csum = _run_ae(ea_packed, g0_We, g0_ae.reshape(-1, 1),
                        g1_We, g1_ae.reshape(-1, 1))
    ae0 = aep[:, 0:8]
    ae1 = aep[:, 8:16]

    pad = E_PAD - E
    pad_idx = (jnp.arange(pad, dtype=jnp.int32) % N)
    src_p = jnp.concatenate([edge_index[0], pad_idx])
    dst_p = jnp.concatenate([edge_index[1], pad_idx])
    neg = jnp.full((pad,), -1e30, jnp.float32)
    ae0_p = jnp.concatenate([ae0.reshape(-1), neg])
    ae1_p = jnp.concatenate([ae1.reshape(-1), neg])

    xp = jnp.concatenate([x, jnp.zeros((NP - N, FDIM), jnp.float32)], axis=0)
    batch_p = jnp.concatenate(
        [batch, jnp.full((NP - N,), 16, batch.dtype)]).reshape(-1, 1)

    h0, s0, d0 = _run_node(xp, g0_W, av0T)
    acc0, dpart0 = _run_sc_edge(h0, s0.reshape(-1), d0.reshape(-1),
                                src_p, dst_p, ae0_p)
    h1, s1, d1 = _run_mid(acc0, dpart0, s0, d0, csum, h0,
                          g0_b.reshape(1, -1), g1_W, av1T)
    acc1, dpart1 = _run_sc_edge(h1, s1.reshape(-1), d1.reshape(-1),
                                src_p, dst_p, ae1_p)
    out = _run_head(acc1, dpart1, s1, d1, csum, h1, g1_b.reshape(1, -1),
                    batch_p, lin0_W, lin0_b, h0_W, h0_b, h1_W, h1_b)
    return out


# speedup vs baseline: 37.8665x; 1.0840x over previous
import jax
import jax.numpy as jnp
from jax import lax
from jax.experimental import pallas as pl
from jax.experimental.pallas import tpu as pltpu
from jax.experimental.pallas import tpu_sc as plsc

N_NODES = 10000
NP = 10240
N_EDGES = 320000
FDIM = 128
NC = 2
NS = 16
NW = NC * NS
CH = 96
CHUNKS = 108
EW = CHUNKS * CH
E_PAD = EW * NW
ROWS_PER_TILE = NP // NS
ROW_BLK = 64
NBLK = 2048
NGRID = NP // NBLK


def _ae_body(ea_ref, we0_ref, ae0_ref, we1_ref, ae1_ref,
             out_ref, csum_ref):
    ve0 = jnp.dot(we0_ref[...], ae0_ref[...], preferred_element_type=jnp.float32)
    ve1 = jnp.dot(we1_ref[...], ae1_ref[...], preferred_element_type=jnp.float32)
    ve_cat = jnp.concatenate([ve0, ve1], axis=1)
    l_row = lax.broadcasted_iota(jnp.int32, (128, 16), 0)
    c_col = lax.broadcasted_iota(jnp.int32, (128, 16), 1)
    T = (l_row % 16 == c_col).astype(jnp.float32)
    vb = jnp.dot(T, ve_cat, preferred_element_type=jnp.float32)
    m_pre = jnp.concatenate(
        [jnp.broadcast_to(vb[:, 0:1], (128, 8)),
         jnp.broadcast_to(vb[:, 1:2], (128, 8))], axis=1)
    mask = (l_row // 16 == c_col % 8).astype(jnp.float32)
    M = m_pre * mask
    out = jnp.dot(ea_ref[...], M, preferred_element_type=jnp.float32)
    out_ref[...] = out

    @pl.when(pl.program_id(0) == 0)
    def _():
        csum_ref[...] = jnp.zeros_like(csum_ref)

    s0 = jnp.sum(out[:, 0:8]).reshape(1, 1)
    s1 = jnp.sum(out[:, 8:16]).reshape(1, 1)
    csum_ref[...] += jnp.concatenate([s0, s1], axis=1)


def _run_ae(ea_packed, g0_We, g0_ae_col, g1_We, g1_ae_col):
    R = ea_packed.shape[0]
    BLK = 8000
    grid = (R // BLK,)
    return pl.pallas_call(
        _ae_body,
        grid=grid,
        in_specs=[
            pl.BlockSpec((BLK, 128), lambda i: (i, 0)),
            pl.BlockSpec(g0_We.shape, lambda i: (0, 0)),
            pl.BlockSpec(g0_ae_col.shape, lambda i: (0, 0)),
            pl.BlockSpec(g1_We.shape, lambda i: (0, 0)),
            pl.BlockSpec(g1_ae_col.shape, lambda i: (0, 0)),
        ],
        out_specs=[
            pl.BlockSpec((BLK, 16), lambda i: (i, 0)),
            pl.BlockSpec((1, 2), lambda i: (0, 0)),
        ],
        out_shape=[
            jax.ShapeDtypeStruct((R, 16), jnp.float32),
            jax.ShapeDtypeStruct((1, 2), jnp.float32),
        ],
    )(ea_packed, g0_We, g0_ae_col, g1_We, g1_ae_col)


def _pack_sd(s, d):
    sb = lax.bitcast_convert_type(s, jnp.int32)
    db = lax.bitcast_convert_type(d, jnp.int32)
    hi = (sb + 0x8000) & jnp.int32(-65536)
    lo = lax.shift_right_logical(db + 0x8000, 16)
    return hi | lo


def _node_body(x_ref, w_ref, avt_ref, h_ref, s_ref, d_ref, sdp_ref):
    h = jnp.dot(x_ref[...], w_ref[...], preferred_element_type=jnp.float32)
    h_ref[...] = h
    s = lax.dot_general(h, avt_ref[...][:, 0], (((1,), (0,)), ((), ())),
                        preferred_element_type=jnp.float32)
    d = lax.dot_general(h, avt_ref[...][:, 1], (((1,), (0,)), ((), ())),
                        preferred_element_type=jnp.float32)
    s_ref[...] = s
    d_ref[...] = d
    sdp_ref[...] = _pack_sd(s, d)


def _run_node(x, W, avT):
    return pl.pallas_call(
        _node_body,
        grid=(NGRID,),
        in_specs=[
            pl.BlockSpec((NBLK, FDIM), lambda i: (i, 0)),
            pl.BlockSpec((FDIM, FDIM), lambda i: (0, 0)),
            pl.BlockSpec((FDIM, 2), lambda i: (0, 0)),
        ],
        out_specs=[
            pl.BlockSpec((NBLK, FDIM), lambda i: (i, 0)),
            pl.BlockSpec((NBLK,), lambda i: (i,)),
            pl.BlockSpec((NBLK,), lambda i: (i,)),
            pl.BlockSpec((NBLK,), lambda i: (i,)),
        ],
        out_shape=[
            jax.ShapeDtypeStruct((NP, FDIM), jnp.float32),
            jax.ShapeDtypeStruct((NP,), jnp.float32),
            jax.ShapeDtypeStruct((NP,), jnp.float32),
            jax.ShapeDtypeStruct((NP,), jnp.int32),
        ],
    )(x, W, avT)


def _mid_body(acca_ref, accb_ref, dpart_ref, s_in, d_in, csum_ref, h_ref,
              b_ref, w1_ref, av1t_ref, h1_ref, s1_ref, d1_ref, sdp1_ref):
    cl = csum_ref[0, 0] * (1.0 / N_EDGES)
    al = (s_in[...] + d_in[...] + cl).reshape(-1, 1)
    al = jnp.where(al > 0, al, 0.2 * al)
    exl = jnp.exp(al)
    ones = jnp.ones((NW, 1), jnp.float32)
    dsum = lax.dot_general(dpart_ref[...], ones, (((0,), (0,)), ((), ())),
                           preferred_element_type=jnp.float32)
    rden = 1.0 / (dsum + exl + 1e-16)
    h = h_ref[...]
    out = (acca_ref[...] + accb_ref[...] + exl * h) * rden + b_ref[...]
    x1 = jnp.maximum(out, 0.0)
    h1 = jnp.dot(x1, w1_ref[...], preferred_element_type=jnp.float32)
    h1_ref[...] = h1
    s1 = lax.dot_general(h1, av1t_ref[...][:, 0], (((1,), (0,)), ((), ())),
                         preferred_element_type=jnp.float32)
    d1 = lax.dot_general(h1, av1t_ref[...][:, 1], (((1,), (0,)), ((), ())),
                         preferred_element_type=jnp.float32)
    s1_ref[...] = s1
    d1_ref[...] = d1
    sdp1_ref[...] = _pack_sd(s1, d1)


def _run_mid(acca, accb, dpart, s, d, csum, h, b_row, W1, av1T):
    return pl.pallas_call(
        _mid_body,
        grid=(NGRID,),
        in_specs=[
            pl.BlockSpec((NBLK, FDIM), lambda i: (i, 0)),
            pl.BlockSpec((NBLK, FDIM), lambda i: (i, 0)),
            pl.BlockSpec((NW, NBLK), lambda i: (0, i)),
            pl.BlockSpec((NBLK,), lambda i: (i,)),
            pl.BlockSpec((NBLK,), lambda i: (i,)),
            pl.BlockSpec((1, 2), lambda i: (0, 0)),
            pl.BlockSpec((NBLK, FDIM), lambda i: (i, 0)),
            pl.BlockSpec((1, FDIM), lambda i: (0, 0)),
            pl.BlockSpec((FDIM, FDIM), lambda i: (0, 0)),
            pl.BlockSpec((FDIM, 2), lambda i: (0, 0)),
        ],
        out_specs=[
            pl.BlockSpec((NBLK, FDIM), lambda i: (i, 0)),
            pl.BlockSpec((NBLK,), lambda i: (i,)),
            pl.BlockSpec((NBLK,), lambda i: (i,)),
            pl.BlockSpec((NBLK,), lambda i: (i,)),
        ],
        out_shape=[
            jax.ShapeDtypeStruct((NP, FDIM), jnp.float32),
            jax.ShapeDtypeStruct((NP,), jnp.float32),
            jax.ShapeDtypeStruct((NP,), jnp.float32),
            jax.ShapeDtypeStruct((NP,), jnp.int32),
        ],
    )(acca, accb, dpart, s, d, csum, h, b_row, W1, av1T)


def _head_body(acca_ref, accb_ref, dpart_ref, s_in, d_in, csum_ref, h_ref,
               b_ref, batch_ref, lin0w_ref, lin0b_ref, h0w_ref, h0b_ref,
               h1w_ref, h1b_ref, out_ref, z_scr):
    i = pl.program_id(0)
    cl = csum_ref[0, 1] * (1.0 / N_EDGES)
    al = (s_in[...] + d_in[...] + cl).reshape(-1, 1)
    al = jnp.where(al > 0, al, 0.2 * al)
    exl = jnp.exp(al)
    ones = jnp.ones((NW, 1), jnp.float32)
    dsum = lax.dot_general(dpart_ref[...], ones, (((0,), (0,)), ((), ())),
                           preferred_element_type=jnp.float32)
    rden = 1.0 / (dsum + exl + 1e-16)
    hf = (acca_ref[...] + accb_ref[...] + exl * h_ref[...]) * rden + b_ref[...]

    batch = batch_ref[...]
    gids = lax.broadcasted_iota(jnp.int32, (1, 16), 1)
    lt = (batch < gids).astype(jnp.float32)
    onesn = jnp.ones((NP, 1), jnp.float32)
    counts = lax.dot_general(lt, onesn, (((0,), (0,)), ((), ())),
                             preferred_element_type=jnp.float32
                             ).astype(jnp.int32)
    blk_iota = lax.broadcasted_iota(jnp.int32, (16, NBLK), 1) + i * NBLK
    onehot = (blk_iota == counts).astype(jnp.float32)

    @pl.when(i == 0)
    def _():
        z_scr[...] = jnp.zeros_like(z_scr)

    z_scr[...] += jnp.dot(onehot, hf, preferred_element_type=jnp.float32)

    @pl.when(i == NGRID - 1)
    def _():
        z = z_scr[...]
        z = jnp.maximum(jnp.dot(z, lin0w_ref[...],
                                preferred_element_type=jnp.float32)
                        + lin0b_ref[...], 0.0)
        z = jnp.maximum(jnp.dot(z, h0w_ref[...],
                                preferred_element_type=jnp.float32)
                        + h0b_ref[...], 0.0)
        out_ref[...] = jnp.dot(z, h1w_ref[...],
                               preferred_element_type=jnp.float32) + h1b_ref[...]


def _run_head(acca, accb, dpart, s, d, csum, h, b_row, batch_col,
              lin0_W, lin0_b, h0_W, h0_b, h1_W, h1_b):
    return pl.pallas_call(
        _head_body,
        grid=(NGRID,),
        in_specs=[
            pl.BlockSpec((NBLK, FDIM), lambda i: (i, 0)),
            pl.BlockSpec((NBLK, FDIM), lambda i: (i, 0)),
            pl.BlockSpec((NW, NBLK), lambda i: (0, i)),
            pl.BlockSpec((NBLK,), lambda i: (i,)),
            pl.BlockSpec((NBLK,), lambda i: (i,)),
            pl.BlockSpec((1, 2), lambda i: (0, 0)),
            pl.BlockSpec((NBLK, FDIM), lambda i: (i, 0)),
            pl.BlockSpec((1, FDIM), lambda i: (0, 0)),
            pl.BlockSpec((NP, 1), lambda i: (0, 0)),
            pl.BlockSpec((FDIM, FDIM), lambda i: (0, 0)),
            pl.BlockSpec((1, FDIM), lambda i: (0, 0)),
            pl.BlockSpec((FDIM, 64), lambda i: (0, 0)),
            pl.BlockSpec((1, 64), lambda i: (0, 0)),
            pl.BlockSpec((64, 16), lambda i: (0, 0)),
            pl.BlockSpec((1, 16), lambda i: (0, 0)),
        ],
        out_specs=pl.BlockSpec((16, 16), lambda i: (0, 0)),
        out_shape=jax.ShapeDtypeStruct((16, 16), jnp.float32),
        scratch_shapes=[pltpu.VMEM((16, FDIM), jnp.float32)],
    )(acca, accb, dpart, s, d, csum, h, b_row, batch_col,
      lin0_W, lin0_b.reshape(1, -1), h0_W, h0_b.reshape(1, -1),
      h1_W, h1_b.reshape(1, -1))


def _sc_edge_body(h_hbm, sdp_hbm, src_hbm, dst_hbm, ae_hbm,
                  acca_hbm, accb_hbm, dpart_hbm,
                  sd_v, den_v,
                  sv0, sv1, sv2, dv0, dv1, dv2, av0, av1, av2,
                  rows0, rows1, ex_v,
                  acc_sh,
                  semi0, semi1, semi2, semg0, semg1, semsc):
    cid = lax.axis_index("c")
    sid = lax.axis_index("s")
    wid = cid * NS + sid
    wbase = wid * EW

    svs = (sv0, sv1, sv2)
    dvs = (dv0, dv1, dv2)
    avs = (av0, av1, av2)
    semis = (semi0, semi1, semi2)
    rows = (rows0, rows1)
    semgs = (semg0, semg1)

    pltpu.sync_copy(sdp_hbm, sd_v)

    zero16 = jnp.zeros((16,), jnp.float32)

    @plsc.parallel_loop(0, NP // 16, unroll=4)
    def _(i):
        den_v[pl.ds(i * 16, 16)] = zero16

    @plsc.parallel_loop(0, CH, unroll=4)
    def _(i):
        for k in range(FDIM // 16):
            rows0[i, pl.ds(k * 16, 16)] = zero16

    for j in range(6):
        pltpu.sync_copy(rows0,
                        acc_sh.at[pl.ds(sid * ROWS_PER_TILE + j * CH, CH)])
    pltpu.sync_copy(rows0.at[pl.ds(0, 64)],
                    acc_sh.at[pl.ds(sid * ROWS_PER_TILE + 6 * CH, 64)])
    plsc.subcore_barrier()

    def _stage(i, r):
        base = wbase + i * CH
        pltpu.async_copy(src_hbm.at[pl.ds(base, CH)], svs[r], semis[r])
        pltpu.async_copy(dst_hbm.at[pl.ds(base, CH)], dvs[r], semis[r])
        pltpu.async_copy(ae_hbm.at[pl.ds(base, CH)], avs[r], semis[r])

    def _stage_wait(i, r):
        base = wbase + i * CH
        pltpu.make_async_copy(src_hbm.at[pl.ds(base, CH)], svs[r],
                              semis[r]).wait()
        pltpu.make_async_copy(dst_hbm.at[pl.ds(base, CH)], dvs[r],
                              semis[r]).wait()
        pltpu.make_async_copy(ae_hbm.at[pl.ds(base, CH)], avs[r],
                              semis[r]).wait()

    def _gather(r, b):
        pltpu.async_copy(h_hbm.at[svs[r]], rows[b], semgs[b])

    _stage(0, 0)
    _stage(1, 1)
    _stage_wait(0, 0)
    _gather(0, 0)

    def _chunk(i, r, b, guard):
        rows_b = rows[b]
        rp = (r + 2) % 3

        if guard:
            @pl.when(i > 0)
            def _():
                pltpu.make_async_copy(rows[1 - b],
                                      acc_sh.at[dvs[rp]], semsc).wait()
        else:
            pltpu.make_async_copy(rows[1 - b],
                                  acc_sh.at[dvs[rp]], semsc).wait()

        @pl.when(i + 2 < CHUNKS)
        def _():
            _stage(i + 2, rp)

        @pl.when(i + 1 < CHUNKS)
        def _():
            rn = (r + 1) % 3
            _stage_wait(i + 1, rn)
            _gather(rn, 1 - b)

        pltpu.make_async_copy(h_hbm.at[svs[r]], rows_b, semgs[b]).wait()

        for g in range(CH // 16):
            sl = pl.ds(g * 16, 16)
            sv = svs[r][sl]
            dv = dvs[r][sl]
            ps = plsc.load_gather(sd_v, [sv])
            pd = plsc.load_gather(sd_v, [dv])
            s_f = plsc.bitcast(ps & jnp.int32(-65536), jnp.float32)
            d_f = plsc.bitcast(lax.shift_left(pd, 16), jnp.float32)
            a = s_f + d_f + avs[r][sl]
            a = jnp.where(a > 0, a, 0.2 * a)
            ex = jnp.exp(a)
            plsc.addupdate_scatter(den_v, [dv], ex)
            ex_v[sl] = ex

        @plsc.parallel_loop(0, CH, unroll=4)
        def _(e):
            eidx = jnp.broadcast_to(e, (16,)).astype(jnp.int32)
            ce = plsc.load_gather(ex_v, [eidx])
            for k in range(FDIM // 16):
                ksl = pl.ds(k * 16, 16)
                rows_b[e, ksl] = rows_b[e, ksl] * ce

        pltpu.async_copy(rows_b, acc_sh.at[dvs[r]], semsc, add=True)

    def _six(it, carry):
        i0 = it * 6
        for u in range(6):
            _chunk(i0 + u, u % 3, u % 2, u == 0)
        return carry

    lax.fori_loop(0, CHUNKS // 6, _six, 0)

    pltpu.make_async_copy(rows[(CHUNKS - 1) % 2],
                          acc_sh.at[dvs[(CHUNKS - 1) % 3]],
                          semsc).wait()

    pltpu.sync_copy(den_v, dpart_hbm.at[wid])
    plsc.subcore_barrier()
    row_sl = pl.ds(sid * ROWS_PER_TILE, ROWS_PER_TILE)

    @pl.when(cid == 0)
    def _():
        pltpu.sync_copy(acc_sh.at[row_sl], acca_hbm.at[row_sl])

    @pl.when(cid == 1)
    def _():
        pltpu.sync_copy(acc_sh.at[row_sl], accb_hbm.at[row_sl])


def _run_sc_edge(h, sdp, src_p, dst_p, ae_p):
    mesh = plsc.VectorSubcoreMesh(core_axis_name="c", subcore_axis_name="s")
    fn = pl.kernel(
        _sc_edge_body,
        out_type=[
            jax.ShapeDtypeStruct((NP, FDIM), jnp.float32),
            jax.ShapeDtypeStruct((NP, FDIM), jnp.float32),
            jax.ShapeDtypeStruct((NW, NP), jnp.float32),
        ],
        mesh=mesh,
        compiler_params=pltpu.CompilerParams(needs_layout_passes=False),
        scratch_types=[
            pltpu.VMEM((NP,), jnp.int32),
            pltpu.VMEM((NP,), jnp.float32),
            pltpu.VMEM((CH,), jnp.int32),
            pltpu.VMEM((CH,), jnp.int32),
            pltpu.VMEM((CH,), jnp.int32),
            pltpu.VMEM((CH,), jnp.int32),
            pltpu.VMEM((CH,), jnp.int32),
            pltpu.VMEM((CH,), jnp.int32),
            pltpu.VMEM((CH,), jnp.float32),
            pltpu.VMEM((CH,), jnp.float32),
            pltpu.VMEM((CH,), jnp.float32),
            pltpu.VMEM((CH, FDIM), jnp.float32),
            pltpu.VMEM((CH, FDIM), jnp.float32),
            pltpu.VMEM((CH,), jnp.float32),
            pltpu.VMEM_SHARED((NP, FDIM), jnp.float32),
            pltpu.SemaphoreType.DMA,
            pltpu.SemaphoreType.DMA,
            pltpu.SemaphoreType.DMA,
            pltpu.SemaphoreType.DMA,
            pltpu.SemaphoreType.DMA,
            pltpu.SemaphoreType.DMA,
        ],
    )
    return fn(h, sdp, src_p, dst_p, ae_p)


def kernel(x, edge_index, batch, edge_attr,
           g0_W, g0_asrc, g0_adst, g0_We, g0_ae, g0_b,
           g1_W, g1_asrc, g1_adst, g1_We, g1_ae, g1_b,
           lin0_W, lin0_b, h0_W, h0_b, h1_W, h1_b):
    N = x.shape[0]
    E = edge_index.shape[1]

    av0T = jnp.stack([g0_asrc[0], g0_adst[0]], axis=1)
    av1T = jnp.stack([g1_asrc[0], g1_adst[0]], axis=1)

    ea_packed = edge_attr.reshape(E // 8, 8 * edge_attr.shape[1])
    aep, csum = _run_ae(ea_packed, g0_We, g0_ae.reshape(-1, 1),
                        g1_We, g1_ae.reshape(-1, 1))
    ae0 = aep[:, 0:8]
    ae1 = aep[:, 8:16]

    pad = E_PAD - E
    pad_idx = (jnp.arange(pad, dtype=jnp.int32) % N)
    src_p = jnp.concatenate([edge_index[0], pad_idx])
    dst_p = jnp.concatenate([edge_index[1], pad_idx])
    neg = jnp.full((pad,), -1e30, jnp.float32)
    ae0_p = jnp.concatenate([ae0.reshape(-1), neg])
    ae1_p = jnp.concatenate([ae1.reshape(-1), neg])

    xp = jnp.concatenate([x, jnp.zeros((NP - N, FDIM), jnp.float32)], axis=0)
    batch_p = jnp.concatenate(
        [batch, jnp.full((NP - N,), 16, batch.dtype)]).reshape(-1, 1)

    h0, s0, d0, sdp0 = _run_node(xp, g0_W, av0T)
    acca0, accb0, dpart0 = _run_sc_edge(h0, sdp0, src_p, dst_p, ae0_p)
    h1, s1, d1, sdp1 = _run_mid(acca0, accb0, dpart0, s0, d0, csum, h0,
                                g0_b.reshape(1, -1), g1_W, av1T)
    acca1, accb1, dpart1 = _run_sc_edge(h1, sdp1, src_p, dst_p, ae1_p)
    out = _run_head(acca1, accb1, dpart1, s1, d1, csum, h1,
                    g1_b.reshape(1, -1), batch_p,
                    lin0_W, lin0_b, h0_W, h0_b, h1_W, h1_b)
    return out
